# Initial kernel scaffold; baseline (speedup 1.0000x reference)
import functools

import jax
import jax.numpy as jnp
from jax import lax
from jax.experimental import pallas as pl
from jax.experimental.pallas import tpu as pltpu
from jax.experimental.pallas import tpu_sc as plsc

VD = 32
HD = 64
N = 10000
E = 160000
DU = 160
DO = 176

NC, NS = 2, 16
NW = NC * NS
EPW = E // NW
CH = 128

BE = 800
BN = 1000


def _edge_core(F0, F1, F2, sp, w1f, b1, w2, b2, w3, b3):
    B = F0.shape[0]
    Fs = (F0, F1, F2)
    til = [jnp.concatenate([Fk] * 4, axis=1) for Fk in Fs]
    reps = [[None] * 8 for _ in range(3)]
    blocks = []
    for c in range(8):
        blk = None
        for k in range(3):
            Fk = Fs[k]
            rep = jnp.concatenate(
                [jnp.broadcast_to(Fk[:, 4 * c + t:4 * c + t + 1], (B, 32))
                 for t in range(4)], axis=1)
            reps[k][c] = rep
            term = rep * til[k]
            blk = term if blk is None else blk + term
        blocks.append(blk)
    f2s = jnp.concatenate(blocks, axis=1)
    nrm2 = None
    for blk in blocks:
        part = jnp.sum(blk * blk, axis=1, keepdims=True)
        nrm2 = part if nrm2 is None else nrm2 + part
    fnorm = jnp.sqrt(nrm2) + 1.0

    h = jnp.dot(f2s, w1f, preferred_element_type=jnp.float32) + sp + b1
    h = jnp.maximum(h, 0.0)
    h = jnp.dot(h, w2, preferred_element_type=jnp.float32) + b2
    h = jnp.maximum(h, 0.0)
    cvec = jnp.dot(h, w3, preferred_element_type=jnp.float32) + b3
    cvec = cvec * (1.0 / fnorm)

    outs = []
    for k in range(3):
        acc = None
        for c in range(8):
            term = reps[k][c] * cvec[:, 128 * c:128 * (c + 1)]
            acc = term if acc is None else acc + term
        out_k = (acc[:, 0:32] + acc[:, 32:64]
                 + acc[:, 64:96] + acc[:, 96:128])
        outs.append(out_k)
    f_msg = jnp.concatenate(outs, axis=1)
    s_msg = cvec[:, 1024:1088]
    return f_msg, s_msg


def _prep_body(f_ref, s_ref, wa_ref, wb_ref, ws1_ref, ws2_ref, u_ref, v_ref):
    fb = f_ref[...]
    sb = s_ref[...]
    a = jnp.dot(fb, wa_ref[...], preferred_element_type=jnp.float32)
    b = jnp.dot(fb, wb_ref[...], preferred_element_type=jnp.float32)
    ps = jnp.dot(sb, ws1_ref[...], preferred_element_type=jnp.float32)
    qs = jnp.dot(sb, ws2_ref[...], preferred_element_type=jnp.float32)
    u_ref[...] = jnp.concatenate([a, ps], axis=1)
    v_ref[...] = jnp.concatenate([b, qs], axis=1)


def _run_prep(f_flat, s, wa96, wb96, w1s1, w1s2):
    nb = N // BN
    full = lambda shp: pl.BlockSpec(shp, lambda i: (0, 0))
    return pl.pallas_call(
        _prep_body,
        grid=(nb,),
        in_specs=[
            pl.BlockSpec((BN, 96), lambda i: (i, 0)),
            pl.BlockSpec((BN, HD), lambda i: (i, 0)),
            full((96, 96)), full((96, 96)), full((HD, HD)), full((HD, HD)),
        ],
        out_specs=[
            pl.BlockSpec((BN, DU), lambda i: (i, 0)),
            pl.BlockSpec((BN, DU), lambda i: (i, 0)),
        ],
        out_shape=[
            jax.ShapeDtypeStruct((N, DU), jnp.float32),
            jax.ShapeDtypeStruct((N, DU), jnp.float32),
        ],
    )(f_flat, s, wa96, wb96, w1s1, w1s2)


def _run_gather(u_tab, v_tab, src, dst):
    mesh = plsc.VectorSubcoreMesh(core_axis_name="c", subcore_axis_name="s")
    nch = (EPW + CH - 1) // CH

    @functools.partial(
        pl.kernel,
        mesh=mesh,
        out_type=[
            jax.ShapeDtypeStruct((E, DU), jnp.float32),
            jax.ShapeDtypeStruct((E, DU), jnp.float32),
        ],
        scratch_types=[
            pltpu.VMEM((CH,), jnp.int32),
            pltpu.VMEM((CH, DU), jnp.float32),
            pltpu.SemaphoreType.DMA,
        ],
    )
    def k(u_hbm, v_hbm, src_hbm, dst_hbm, gu_hbm, gv_hbm, idx_v, rows_v, sem):
        cid = lax.axis_index("c")
        sid = lax.axis_index("s")
        base_w = cid * (E // NC) + sid * EPW

        def body(ci, carry):
            base = jnp.minimum(base_w + ci * CH, E - CH)
            pltpu.sync_copy(src_hbm.at[pl.ds(base, CH)], idx_v)
            pltpu.async_copy(u_hbm.at[idx_v], rows_v, sem).wait()
            pltpu.sync_copy(rows_v, gu_hbm.at[pl.ds(base, CH)])
            pltpu.sync_copy(dst_hbm.at[pl.ds(base, CH)], idx_v)
            pltpu.async_copy(v_hbm.at[idx_v], rows_v, sem).wait()
            pltpu.sync_copy(rows_v, gv_hbm.at[pl.ds(base, CH)])
            return carry

        lax.fori_loop(0, nch, body, 0)

    return k(u_tab, v_tab, src, dst)


def _edge_body(gu_ref, gv_ref, w1f_ref, b1_ref, w2_ref, b2_ref, w3_ref,
               b3_ref, o_ref):
    u = gu_ref[...]
    v = gv_ref[...]
    fvec = u[:, :96] + v[:, :96]
    sp = u[:, 96:] + v[:, 96:]
    f_msg, s_msg = _edge_core(
        fvec[:, 0:32], fvec[:, 32:64], fvec[:, 64:96], sp,
        w1f_ref[...], b1_ref[...], w2_ref[...], b2_ref[...],
        w3_ref[...], b3_ref[...])
    ones = jnp.ones((f_msg.shape[0], 16), jnp.float32)
    o_ref[...] = jnp.concatenate([f_msg, s_msg, ones], axis=1)


def _run_edge(gu, gv, w1f, b1, w2, b2, w3, b3):
    nb = E // BE
    full = lambda shp: pl.BlockSpec(shp, lambda i: tuple(0 for _ in shp))
    return pl.pallas_call(
        _edge_body,
        grid=(nb,),
        in_specs=[
            pl.BlockSpec((BE, DU), lambda i: (i, 0)),
            pl.BlockSpec((BE, DU), lambda i: (i, 0)),
            full((1024, HD)), full((1, HD)), full((HD, HD)), full((1, HD)),
            full((HD, 1088)), full((1, 1088)),
        ],
        out_specs=pl.BlockSpec((BE, DO), lambda i: (i, 0)),
        out_shape=jax.ShapeDtypeStruct((E, DO), jnp.float32),
    )(gu, gv, w1f, b1, w2, b2, w3, b3)


def _run_scatter(o_rows, src, zeros_init):
    mesh = plsc.VectorSubcoreMesh(core_axis_name="c", subcore_axis_name="s")
    nfull = EPW // CH
    tail = EPW - nfull * CH
    rows_per_tile = N // NS

    @functools.partial(
        pl.kernel,
        mesh=mesh,
        out_type=jax.ShapeDtypeStruct((NC, N, DO), jnp.float32),
        scratch_types=[
            pltpu.VMEM((CH,), jnp.int32),
            pltpu.VMEM((CH, DO), jnp.float32),
            pltpu.VMEM((tail,), jnp.int32),
            pltpu.VMEM((tail, DO), jnp.float32),
            pltpu.VMEM_SHARED((N, DO), jnp.float32),
        ],
    )
    def k(o_hbm, src_hbm, z_hbm, out_hbm, idx_v, rows_v, idx_t, rows_t, accum):
        cid = lax.axis_index("c")
        sid = lax.axis_index("s")
        base_w = cid * (E // NC) + sid * EPW
        r0 = sid * rows_per_tile

        pltpu.sync_copy(z_hbm, accum.at[pl.ds(r0, rows_per_tile)])
        plsc.subcore_barrier()

        def body(ci, carry):
            base = base_w + ci * CH
            pltpu.sync_copy(src_hbm.at[pl.ds(base, CH)], idx_v)
            pltpu.sync_copy(o_hbm.at[pl.ds(base, CH)], rows_v)
            pltpu.sync_copy(rows_v, accum.at[idx_v], add=True)
            return carry

        lax.fori_loop(0, nfull, body, 0)

        bt = base_w + nfull * CH
        pltpu.sync_copy(src_hbm.at[pl.ds(bt, tail)], idx_t)
        pltpu.sync_copy(o_hbm.at[pl.ds(bt, tail)], rows_t)
        pltpu.sync_copy(rows_t, accum.at[idx_t], add=True)

        plsc.subcore_barrier()
        pltpu.sync_copy(accum.at[pl.ds(r0, rows_per_tile)],
                        out_hbm.at[cid, pl.ds(r0, rows_per_tile)])

    return k(o_rows, src, zeros_init)


def _node_body(f_ref, s_ref, p_ref, wa_ref, wb_ref, w1f_ref, ws1_ref,
               ws2_ref, b1_ref, w2_ref, b2_ref, w3_ref, b3_ref,
               fo_ref, so_ref):
    fb = f_ref[...]
    sb = s_ref[...]
    ps = p_ref[0] + p_ref[1]
    cnt = jnp.maximum(ps[:, 160:161], 1.0)
    inv = 1.0 / cnt
    f_c = ps[:, :96] * inv
    s_c = ps[:, 96:160] * inv

    tf = (jnp.dot(fb, wa_ref[...], preferred_element_type=jnp.float32)
          + jnp.dot(f_c, wb_ref[...], preferred_element_type=jnp.float32))
    sp = (jnp.dot(sb, ws1_ref[...], preferred_element_type=jnp.float32)
          + jnp.dot(s_c, ws2_ref[...], preferred_element_type=jnp.float32))
    f_msg, s_msg = _edge_core(
        tf[:, 0:32], tf[:, 32:64], tf[:, 64:96], sp,
        w1f_ref[...], b1_ref[...], w2_ref[...], b2_ref[...],
        w3_ref[...], b3_ref[...])
    fo_ref[...] = f_msg + fb
    so_ref[...] = s_msg + sb


def _run_node(f_flat, s, partials, wa2, wb2, sw1f, sw1s, sw1sc, sb1, sw2,
              sb2, sw3, sb3):
    nb = N // BN
    full = lambda shp: pl.BlockSpec(shp, lambda i: tuple(0 for _ in shp))
    return pl.pallas_call(
        _node_body,
        grid=(nb,),
        in_specs=[
            pl.BlockSpec((BN, 96), lambda i: (i, 0)),
            pl.BlockSpec((BN, HD), lambda i: (i, 0)),
            pl.BlockSpec((NC, BN, DO), lambda i: (0, i, 0)),
            full((96, 96)), full((96, 96)),
            full((1024, HD)), full((HD, HD)), full((HD, HD)), full((1, HD)),
            full((HD, HD)), full((1, HD)), full((HD, 1088)), full((1, 1088)),
        ],
        out_specs=[
            pl.BlockSpec((BN, 96), lambda i: (i, 0)),
            pl.BlockSpec((BN, HD), lambda i: (i, 0)),
        ],
        out_shape=[
            jax.ShapeDtypeStruct((N, 96), jnp.float32),
            jax.ShapeDtypeStruct((N, HD), jnp.float32),
        ],
    )(f_flat, s, partials, wa2, wb2, sw1f, sw1s, sw1sc, sb1, sw2, sb2,
      sw3, sb3)


def kernel(f, s, edge_index, W_emb1, W_emb2, nW1, nb1, nW2, nb2, nW3, nb3,
           sW1, sb1, sW2, sb2, sW3, sb3):
    f_flat = f.reshape(N, 96)
    src = edge_index[0].astype(jnp.int32)
    dst = edge_index[1].astype(jnp.int32)

    eye3 = jnp.eye(3, dtype=jnp.float32)
    wa96 = jnp.kron(eye3, W_emb1[:VD])
    wb96 = jnp.kron(eye3, W_emb1[VD:])
    wa2 = jnp.kron(eye3, W_emb2[:VD])
    wb2 = jnp.kron(eye3, W_emb2[VD:])

    u_tab, v_tab = _run_prep(f_flat, s, wa96, wb96,
                             nW1[1024:1088], nW1[1088:1152])
    gu, gv = _run_gather(u_tab, v_tab, src, dst)
    o_rows = _run_edge(gu, gv, nW1[:1024], nb1.reshape(1, HD), nW2,
                       nb2.reshape(1, HD), nW3, nb3.reshape(1, 1088))
    zeros_init = jnp.zeros((N // NS, DO), jnp.float32)
    partials = _run_scatter(o_rows, src, zeros_init)
    f_out, s_out = _run_node(f_flat, s, partials, wa2, wb2,
                             sW1[:1024], sW1[1024:1088], sW1[1088:1152],
                             sb1.reshape(1, HD), sW2, sb2.reshape(1, HD),
                             sW3, sb3.reshape(1, 1088))
    return (f_out.reshape(N, 3, VD), s_out)


# speedup vs baseline: 5.6333x; 5.6333x over previous
import functools

import jax
import jax.numpy as jnp
from jax import lax
from jax.experimental import pallas as pl
from jax.experimental.pallas import tpu as pltpu
from jax.experimental.pallas import tpu_sc as plsc

VD = 32
HD = 64
N = 10000
E = 160000
DU = 160
DO = 160
DC = 16

NC, NS = 2, 16
NW = NC * NS
EPW = E // NW
CH = 128

BE = 800
BN = 1000


def _edge_core(F0, F1, F2, sp, w1f, b1, w2, b2, w3, b3):
    B = F0.shape[0]
    Fs = (F0, F1, F2)
    til = [jnp.concatenate([Fk] * 4, axis=1) for Fk in Fs]
    reps = [[None] * 8 for _ in range(3)]
    blocks = []
    for c in range(8):
        blk = None
        for k in range(3):
            Fk = Fs[k]
            rep = jnp.concatenate(
                [jnp.broadcast_to(Fk[:, 4 * c + t:4 * c + t + 1], (B, 32))
                 for t in range(4)], axis=1)
            reps[k][c] = rep
            term = rep * til[k]
            blk = term if blk is None else blk + term
        blocks.append(blk)
    f2s = jnp.concatenate(blocks, axis=1)
    nrm2 = None
    for blk in blocks:
        part = jnp.sum(blk * blk, axis=1, keepdims=True)
        nrm2 = part if nrm2 is None else nrm2 + part
    fnorm = jnp.sqrt(nrm2) + 1.0

    h = jnp.dot(f2s, w1f, preferred_element_type=jnp.float32) + sp + b1
    h = jnp.maximum(h, 0.0)
    h = jnp.dot(h, w2, preferred_element_type=jnp.float32) + b2
    h = jnp.maximum(h, 0.0)
    cvec = jnp.dot(h, w3, preferred_element_type=jnp.float32) + b3
    cvec = cvec * (1.0 / fnorm)

    outs = []
    for k in range(3):
        acc = None
        for c in range(8):
            term = reps[k][c] * cvec[:, 128 * c:128 * (c + 1)]
            acc = term if acc is None else acc + term
        out_k = (acc[:, 0:32] + acc[:, 32:64]
                 + acc[:, 64:96] + acc[:, 96:128])
        outs.append(out_k)
    f_msg = jnp.concatenate(outs, axis=1)
    s_msg = cvec[:, 1024:1088]
    return f_msg, s_msg


def _prep_body(f_ref, s_ref, wa_ref, wb_ref, ws1_ref, ws2_ref, u_ref, v_ref):
    fb = f_ref[...]
    sb = s_ref[...]
    a = jnp.dot(fb, wa_ref[...], preferred_element_type=jnp.float32)
    b = jnp.dot(fb, wb_ref[...], preferred_element_type=jnp.float32)
    ps = jnp.dot(sb, ws1_ref[...], preferred_element_type=jnp.float32)
    qs = jnp.dot(sb, ws2_ref[...], preferred_element_type=jnp.float32)
    u_ref[...] = jnp.concatenate([a, ps], axis=1)
    v_ref[...] = jnp.concatenate([b, qs], axis=1)


def _run_prep(f_flat, s, wa96, wb96, w1s1, w1s2):
    nb = N // BN
    full = lambda shp: pl.BlockSpec(shp, lambda i: (0, 0))
    return pl.pallas_call(
        _prep_body,
        grid=(nb,),
        in_specs=[
            pl.BlockSpec((BN, 96), lambda i: (i, 0)),
            pl.BlockSpec((BN, HD), lambda i: (i, 0)),
            full((96, 96)), full((96, 96)), full((HD, HD)), full((HD, HD)),
        ],
        out_specs=[
            pl.BlockSpec((BN, DU), lambda i: (i, 0)),
            pl.BlockSpec((BN, DU), lambda i: (i, 0)),
        ],
        out_shape=[
            jax.ShapeDtypeStruct((N, DU), jnp.float32),
            jax.ShapeDtypeStruct((N, DU), jnp.float32),
        ],
    )(f_flat, s, wa96, wb96, w1s1, w1s2)


def _run_gather(u_tab, v_tab, src, dst, zeros_c, ones_rows):
    mesh = plsc.VectorSubcoreMesh(core_axis_name="c", subcore_axis_name="s")
    nfull = EPW // CH
    tail = EPW - nfull * CH
    rows_per_tile = N // NS

    @functools.partial(
        pl.kernel,
        mesh=mesh,
        compiler_params=pltpu.CompilerParams(use_tc_tiling_on_sc=False),
        out_type=[
            jax.ShapeDtypeStruct((E, DU), jnp.float32),
            jax.ShapeDtypeStruct((E, DU), jnp.float32),
            jax.ShapeDtypeStruct((NC, N, DC), jnp.float32),
        ],
        scratch_types=[
            pltpu.VMEM((CH,), jnp.int32),
            pltpu.VMEM((CH, DU), jnp.float32),
            pltpu.VMEM((CH, DC), jnp.float32),
            pltpu.VMEM((tail,), jnp.int32),
            pltpu.VMEM((tail, DU), jnp.float32),
            pltpu.VMEM((tail, DC), jnp.float32),
            pltpu.VMEM_SHARED((N, DC), jnp.float32),
            pltpu.SemaphoreType.DMA,
        ],
    )
    def k(u_hbm, v_hbm, src_hbm, dst_hbm, zc_hbm, ones_hbm,
          gu_hbm, gv_hbm, cnt_hbm,
          idx_v, rows_v, ones_v, idx_t, rows_t, ones_t, cacc, sem):
        cid = lax.axis_index("c")
        sid = lax.axis_index("s")
        base_w = cid * (E // NC) + sid * EPW
        r0 = sid * rows_per_tile

        pltpu.sync_copy(ones_hbm, ones_v)
        pltpu.sync_copy(ones_hbm.at[pl.ds(0, tail)], ones_t)
        pltpu.sync_copy(zc_hbm, cacc.at[pl.ds(r0, rows_per_tile)])
        plsc.subcore_barrier()

        def body(ci, carry):
            base = base_w + ci * CH
            pltpu.sync_copy(src_hbm.at[pl.ds(base, CH)], idx_v)
            pltpu.async_copy(u_hbm.at[idx_v], rows_v, sem).wait()
            pltpu.sync_copy(rows_v, gu_hbm.at[pl.ds(base, CH)])
            pltpu.sync_copy(ones_v, cacc.at[idx_v], add=True)
            pltpu.sync_copy(dst_hbm.at[pl.ds(base, CH)], idx_v)
            pltpu.async_copy(v_hbm.at[idx_v], rows_v, sem).wait()
            pltpu.sync_copy(rows_v, gv_hbm.at[pl.ds(base, CH)])
            return carry

        lax.fori_loop(0, nfull, body, 0)

        bt = base_w + nfull * CH
        pltpu.sync_copy(src_hbm.at[pl.ds(bt, tail)], idx_t)
        pltpu.async_copy(u_hbm.at[idx_t], rows_t, sem).wait()
        pltpu.sync_copy(rows_t, gu_hbm.at[pl.ds(bt, tail)])
        pltpu.sync_copy(ones_t, cacc.at[idx_t], add=True)
        pltpu.sync_copy(dst_hbm.at[pl.ds(bt, tail)], idx_t)
        pltpu.async_copy(v_hbm.at[idx_t], rows_t, sem).wait()
        pltpu.sync_copy(rows_t, gv_hbm.at[pl.ds(bt, tail)])

        plsc.subcore_barrier()
        pltpu.sync_copy(cacc.at[pl.ds(r0, rows_per_tile)],
                        cnt_hbm.at[cid, pl.ds(r0, rows_per_tile)])

    return k(u_tab, v_tab, src, dst, zeros_c, ones_rows)


def _edge_body(gu_ref, gv_ref, w1f_ref, b1_ref, w2_ref, b2_ref, w3_ref,
               b3_ref, o_ref):
    u = gu_ref[...]
    v = gv_ref[...]
    fvec = u[:, :96] + v[:, :96]
    sp = u[:, 96:] + v[:, 96:]
    f_msg, s_msg = _edge_core(
        fvec[:, 0:32], fvec[:, 32:64], fvec[:, 64:96], sp,
        w1f_ref[...], b1_ref[...], w2_ref[...], b2_ref[...],
        w3_ref[...], b3_ref[...])
    o_ref[...] = jnp.concatenate([f_msg, s_msg], axis=1)


def _run_edge(gu, gv, w1f, b1, w2, b2, w3, b3):
    nb = E // BE
    full = lambda shp: pl.BlockSpec(shp, lambda i: tuple(0 for _ in shp))
    return pl.pallas_call(
        _edge_body,
        grid=(nb,),
        in_specs=[
            pl.BlockSpec((BE, DU), lambda i: (i, 0)),
            pl.BlockSpec((BE, DU), lambda i: (i, 0)),
            full((1024, HD)), full((1, HD)), full((HD, HD)), full((1, HD)),
            full((HD, 1088)), full((1, 1088)),
        ],
        out_specs=pl.BlockSpec((BE, DO), lambda i: (i, 0)),
        out_shape=jax.ShapeDtypeStruct((E, DO), jnp.float32),
    )(gu, gv, w1f, b1, w2, b2, w3, b3)


def _run_scatter(o_rows, src, zeros_init):
    mesh = plsc.VectorSubcoreMesh(core_axis_name="c", subcore_axis_name="s")
    nfull = EPW // CH
    tail = EPW - nfull * CH
    rows_per_tile = N // NS

    @functools.partial(
        pl.kernel,
        mesh=mesh,
        compiler_params=pltpu.CompilerParams(use_tc_tiling_on_sc=False),
        out_type=jax.ShapeDtypeStruct((NC, N, DO), jnp.float32),
        scratch_types=[
            pltpu.VMEM((CH,), jnp.int32),
            pltpu.VMEM((CH, DO), jnp.float32),
            pltpu.VMEM((tail,), jnp.int32),
            pltpu.VMEM((tail, DO), jnp.float32),
            pltpu.VMEM_SHARED((N, DO), jnp.float32),
        ],
    )
    def k(o_hbm, src_hbm, z_hbm, out_hbm, idx_v, rows_v, idx_t, rows_t, accum):
        cid = lax.axis_index("c")
        sid = lax.axis_index("s")
        base_w = cid * (E // NC) + sid * EPW
        r0 = sid * rows_per_tile

        pltpu.sync_copy(z_hbm, accum.at[pl.ds(r0, rows_per_tile)])
        plsc.subcore_barrier()

        def body(ci, carry):
            base = base_w + ci * CH
            pltpu.sync_copy(src_hbm.at[pl.ds(base, CH)], idx_v)
            pltpu.sync_copy(o_hbm.at[pl.ds(base, CH)], rows_v)
            pltpu.sync_copy(rows_v, accum.at[idx_v], add=True)
            return carry

        lax.fori_loop(0, nfull, body, 0)

        bt = base_w + nfull * CH
        pltpu.sync_copy(src_hbm.at[pl.ds(bt, tail)], idx_t)
        pltpu.sync_copy(o_hbm.at[pl.ds(bt, tail)], rows_t)
        pltpu.sync_copy(rows_t, accum.at[idx_t], add=True)

        plsc.subcore_barrier()
        pltpu.sync_copy(accum.at[pl.ds(r0, rows_per_tile)],
                        out_hbm.at[cid, pl.ds(r0, rows_per_tile)])

    return k(o_rows, src, zeros_init)


def _node_body(f_ref, s_ref, p_ref, c_ref, wa_ref, wb_ref, w1f_ref, ws1_ref,
               ws2_ref, b1_ref, w2_ref, b2_ref, w3_ref, b3_ref,
               fo_ref, so_ref):
    fb = f_ref[...]
    sb = s_ref[...]
    ps = p_ref[0] + p_ref[1]
    cnt = jnp.maximum(c_ref[0, :, :1] + c_ref[1, :, :1], 1.0)
    inv = 1.0 / cnt
    f_c = ps[:, :96] * inv
    s_c = ps[:, 96:160] * inv

    tf = (jnp.dot(fb, wa_ref[...], preferred_element_type=jnp.float32)
          + jnp.dot(f_c, wb_ref[...], preferred_element_type=jnp.float32))
    sp = (jnp.dot(sb, ws1_ref[...], preferred_element_type=jnp.float32)
          + jnp.dot(s_c, ws2_ref[...], preferred_element_type=jnp.float32))
    f_msg, s_msg = _edge_core(
        tf[:, 0:32], tf[:, 32:64], tf[:, 64:96], sp,
        w1f_ref[...], b1_ref[...], w2_ref[...], b2_ref[...],
        w3_ref[...], b3_ref[...])
    fo_ref[...] = f_msg + fb
    so_ref[...] = s_msg + sb


def _run_node(f_flat, s, partials, counts, wa2, wb2, sw1f, sw1s, sw1sc, sb1,
              sw2, sb2, sw3, sb3):
    nb = N // BN
    full = lambda shp: pl.BlockSpec(shp, lambda i: tuple(0 for _ in shp))
    return pl.pallas_call(
        _node_body,
        grid=(nb,),
        in_specs=[
            pl.BlockSpec((BN, 96), lambda i: (i, 0)),
            pl.BlockSpec((BN, HD), lambda i: (i, 0)),
            pl.BlockSpec((NC, BN, DO), lambda i: (0, i, 0)),
            pl.BlockSpec((NC, BN, DC), lambda i: (0, i, 0)),
            full((96, 96)), full((96, 96)),
            full((1024, HD)), full((HD, HD)), full((HD, HD)), full((1, HD)),
            full((HD, HD)), full((1, HD)), full((HD, 1088)), full((1, 1088)),
        ],
        out_specs=[
            pl.BlockSpec((BN, 96), lambda i: (i, 0)),
            pl.BlockSpec((BN, HD), lambda i: (i, 0)),
        ],
        out_shape=[
            jax.ShapeDtypeStruct((N, 96), jnp.float32),
            jax.ShapeDtypeStruct((N, HD), jnp.float32),
        ],
    )(f_flat, s, partials, counts, wa2, wb2, sw1f, sw1s, sw1sc, sb1, sw2,
      sb2, sw3, sb3)


def kernel(f, s, edge_index, W_emb1, W_emb2, nW1, nb1, nW2, nb2, nW3, nb3,
           sW1, sb1, sW2, sb2, sW3, sb3):
    f_flat = f.reshape(N, 96)
    src = edge_index[0].astype(jnp.int32)
    dst = edge_index[1].astype(jnp.int32)

    eye3 = jnp.eye(3, dtype=jnp.float32)
    wa96 = jnp.kron(eye3, W_emb1[:VD])
    wb96 = jnp.kron(eye3, W_emb1[VD:])
    wa2 = jnp.kron(eye3, W_emb2[:VD])
    wb2 = jnp.kron(eye3, W_emb2[VD:])

    u_tab, v_tab = _run_prep(f_flat, s, wa96, wb96,
                             nW1[1024:1088], nW1[1088:1152])
    zeros_c = jnp.zeros((N // NS, DC), jnp.float32)
    ones_rows = jnp.ones((CH, DC), jnp.float32)
    gu, gv, counts = _run_gather(u_tab, v_tab, src, dst, zeros_c, ones_rows)
    o_rows = _run_edge(gu, gv, nW1[:1024], nb1.reshape(1, HD), nW2,
                       nb2.reshape(1, HD), nW3, nb3.reshape(1, 1088))
    zeros_init = jnp.zeros((N // NS, DO), jnp.float32)
    partials = _run_scatter(o_rows, src, zeros_init)
    f_out, s_out = _run_node(f_flat, s, partials, counts, wa2, wb2,
                             sW1[:1024], sW1[1024:1088], sW1[1088:1152],
                             sb1.reshape(1, HD), sW2, sb2.reshape(1, HD),
                             sW3, sb3.reshape(1, 1088))
    return (f_out.reshape(N, 3, VD), s_out)


# speedup vs baseline: 10.9869x; 1.9503x over previous
import functools

import jax
import jax.numpy as jnp
from jax import lax
from jax.experimental import pallas as pl
from jax.experimental.pallas import tpu as pltpu
from jax.experimental.pallas import tpu_sc as plsc

VD = 32
HD = 64
N = 10000
E = 160000
DU = 160
DO = 160
DC = 16

NC, NS = 2, 16
NW = NC * NS
EPW = E // NW
CH = 128

BE = 800
BN = 1000


def _edge_core(F0, F1, F2, sp, w1f, b1, w2, b2, w3, b3, rep_m, til_m):
    Fs = (F0, F1, F2)
    Rs = [jnp.dot(Fk, rep_m, preferred_element_type=jnp.float32) for Fk in Fs]
    Ts = [jnp.dot(Fk, til_m, preferred_element_type=jnp.float32) for Fk in Fs]
    f2s = Rs[0] * Ts[0] + Rs[1] * Ts[1] + Rs[2] * Ts[2]
    nrm2 = jnp.sum(f2s * f2s, axis=1, keepdims=True)
    fnorm = jnp.sqrt(nrm2) + 1.0

    h = jnp.dot(f2s, w1f, preferred_element_type=jnp.float32) + sp + b1
    h = jnp.maximum(h, 0.0)
    h = jnp.dot(h, w2, preferred_element_type=jnp.float32) + b2
    h = jnp.maximum(h, 0.0)
    cvec = jnp.dot(h, w3, preferred_element_type=jnp.float32) + b3
    cvec = cvec * (1.0 / fnorm)

    cmain = cvec[:, :1024]
    outs = []
    for k in range(3):
        p = Rs[k] * cmain
        p = p[:, :512] + p[:, 512:]
        p = p[:, :256] + p[:, 256:]
        p = p[:, :128] + p[:, 128:]
        outs.append(p[:, 0:32] + p[:, 32:64] + p[:, 64:96] + p[:, 96:128])
    f_msg = jnp.concatenate(outs, axis=1)
    s_msg = cvec[:, 1024:1088]
    return f_msg, s_msg


def _onehot_mats():
    col = jnp.arange(1024, dtype=jnp.int32)
    row = jnp.arange(32, dtype=jnp.int32)[:, None]
    rep_m = (col[None, :] // 32 == row).astype(jnp.float32)
    til_m = (col[None, :] % 32 == row).astype(jnp.float32)
    return rep_m, til_m


def _prep_body(f_ref, s_ref, wa_ref, wb_ref, ws1_ref, ws2_ref, u_ref, v_ref):
    fb = f_ref[...]
    sb = s_ref[...]
    a = jnp.dot(fb, wa_ref[...], preferred_element_type=jnp.float32)
    b = jnp.dot(fb, wb_ref[...], preferred_element_type=jnp.float32)
    ps = jnp.dot(sb, ws1_ref[...], preferred_element_type=jnp.float32)
    qs = jnp.dot(sb, ws2_ref[...], preferred_element_type=jnp.float32)
    u_ref[...] = jnp.concatenate([a, ps], axis=1)
    v_ref[...] = jnp.concatenate([b, qs], axis=1)


def _run_prep(f_flat, s, wa96, wb96, w1s1, w1s2):
    nb = N // BN
    full = lambda shp: pl.BlockSpec(shp, lambda i: (0, 0))
    return pl.pallas_call(
        _prep_body,
        grid=(nb,),
        in_specs=[
            pl.BlockSpec((BN, 96), lambda i: (i, 0)),
            pl.BlockSpec((BN, HD), lambda i: (i, 0)),
            full((96, 96)), full((96, 96)), full((HD, HD)), full((HD, HD)),
        ],
        out_specs=[
            pl.BlockSpec((BN, DU), lambda i: (i, 0)),
            pl.BlockSpec((BN, DU), lambda i: (i, 0)),
        ],
        out_shape=[
            jax.ShapeDtypeStruct((N, DU), jnp.float32),
            jax.ShapeDtypeStruct((N, DU), jnp.float32),
        ],
    )(f_flat, s, wa96, wb96, w1s1, w1s2)


def _run_gather(u_tab, v_tab, src, dst, zeros_c, ones_rows):
    mesh = plsc.VectorSubcoreMesh(core_axis_name="c", subcore_axis_name="s")
    nfull = EPW // CH
    tail = EPW - nfull * CH
    rows_per_tile = N // NS

    @functools.partial(
        pl.kernel,
        mesh=mesh,
        compiler_params=pltpu.CompilerParams(use_tc_tiling_on_sc=False),
        out_type=[
            jax.ShapeDtypeStruct((E, DU), jnp.float32),
            jax.ShapeDtypeStruct((E, DU), jnp.float32),
            jax.ShapeDtypeStruct((NC, N, DC), jnp.float32),
        ],
        scratch_types=[
            pltpu.VMEM((CH,), jnp.int32),
            pltpu.VMEM((CH, DU), jnp.float32),
            pltpu.VMEM((CH, DC), jnp.float32),
            pltpu.VMEM((tail,), jnp.int32),
            pltpu.VMEM((tail, DU), jnp.float32),
            pltpu.VMEM((tail, DC), jnp.float32),
            pltpu.VMEM_SHARED((N, DC), jnp.float32),
            pltpu.SemaphoreType.DMA,
        ],
    )
    def k(u_hbm, v_hbm, src_hbm, dst_hbm, zc_hbm, ones_hbm,
          gu_hbm, gv_hbm, cnt_hbm,
          idx_v, rows_v, ones_v, idx_t, rows_t, ones_t, cacc, sem):
        cid = lax.axis_index("c")
        sid = lax.axis_index("s")
        base_w = cid * (E // NC) + sid * EPW
        r0 = sid * rows_per_tile

        pltpu.sync_copy(ones_hbm, ones_v)
        pltpu.sync_copy(ones_hbm.at[pl.ds(0, tail)], ones_t)
        pltpu.sync_copy(zc_hbm, cacc.at[pl.ds(r0, rows_per_tile)])
        plsc.subcore_barrier()

        def body(ci, carry):
            base = base_w + ci * CH
            pltpu.sync_copy(src_hbm.at[pl.ds(base, CH)], idx_v)
            pltpu.async_copy(u_hbm.at[idx_v], rows_v, sem).wait()
            pltpu.sync_copy(rows_v, gu_hbm.at[pl.ds(base, CH)])
            pltpu.sync_copy(ones_v, cacc.at[idx_v], add=True)
            pltpu.sync_copy(dst_hbm.at[pl.ds(base, CH)], idx_v)
            pltpu.async_copy(v_hbm.at[idx_v], rows_v, sem).wait()
            pltpu.sync_copy(rows_v, gv_hbm.at[pl.ds(base, CH)])
            return carry

        lax.fori_loop(0, nfull, body, 0)

        bt = base_w + nfull * CH
        pltpu.sync_copy(src_hbm.at[pl.ds(bt, tail)], idx_t)
        pltpu.async_copy(u_hbm.at[idx_t], rows_t, sem).wait()
        pltpu.sync_copy(rows_t, gu_hbm.at[pl.ds(bt, tail)])
        pltpu.sync_copy(ones_t, cacc.at[idx_t], add=True)
        pltpu.sync_copy(dst_hbm.at[pl.ds(bt, tail)], idx_t)
        pltpu.async_copy(v_hbm.at[idx_t], rows_t, sem).wait()
        pltpu.sync_copy(rows_t, gv_hbm.at[pl.ds(bt, tail)])

        plsc.subcore_barrier()
        pltpu.sync_copy(cacc.at[pl.ds(r0, rows_per_tile)],
                        cnt_hbm.at[cid, pl.ds(r0, rows_per_tile)])

    return k(u_tab, v_tab, src, dst, zeros_c, ones_rows)


def _edge_body(gu_ref, gv_ref, w1f_ref, b1_ref, w2_ref, b2_ref, w3_ref,
               b3_ref, rep_ref, til_ref, o_ref):
    u = gu_ref[...]
    v = gv_ref[...]
    fvec = u[:, :96] + v[:, :96]
    sp = u[:, 96:] + v[:, 96:]
    f_msg, s_msg = _edge_core(
        fvec[:, 0:32], fvec[:, 32:64], fvec[:, 64:96], sp,
        w1f_ref[...], b1_ref[...], w2_ref[...], b2_ref[...],
        w3_ref[...], b3_ref[...], rep_ref[...], til_ref[...])
    o_ref[...] = jnp.concatenate([f_msg, s_msg], axis=1)


def _run_edge(gu, gv, w1f, b1, w2, b2, w3, b3, rep_m, til_m):
    nb = E // BE
    full = lambda shp: pl.BlockSpec(shp, lambda i: tuple(0 for _ in shp))
    return pl.pallas_call(
        _edge_body,
        grid=(nb,),
        in_specs=[
            pl.BlockSpec((BE, DU), lambda i: (i, 0)),
            pl.BlockSpec((BE, DU), lambda i: (i, 0)),
            full((1024, HD)), full((1, HD)), full((HD, HD)), full((1, HD)),
            full((HD, 1088)), full((1, 1088)),
            full((VD, 1024)), full((VD, 1024)),
        ],
        out_specs=pl.BlockSpec((BE, DO), lambda i: (i, 0)),
        out_shape=jax.ShapeDtypeStruct((E, DO), jnp.float32),
    )(gu, gv, w1f, b1, w2, b2, w3, b3, rep_m, til_m)


def _run_scatter(o_rows, src, zeros_init):
    mesh = plsc.VectorSubcoreMesh(core_axis_name="c", subcore_axis_name="s")
    nfull = EPW // CH
    tail = EPW - nfull * CH
    rows_per_tile = N // NS

    @functools.partial(
        pl.kernel,
        mesh=mesh,
        compiler_params=pltpu.CompilerParams(use_tc_tiling_on_sc=False),
        out_type=jax.ShapeDtypeStruct((NC, N, DO), jnp.float32),
        scratch_types=[
            pltpu.VMEM((CH,), jnp.int32),
            pltpu.VMEM((CH, DO), jnp.float32),
            pltpu.VMEM((tail,), jnp.int32),
            pltpu.VMEM((tail, DO), jnp.float32),
            pltpu.VMEM_SHARED((N, DO), jnp.float32),
        ],
    )
    def k(o_hbm, src_hbm, z_hbm, out_hbm, idx_v, rows_v, idx_t, rows_t, accum):
        cid = lax.axis_index("c")
        sid = lax.axis_index("s")
        base_w = cid * (E // NC) + sid * EPW
        r0 = sid * rows_per_tile

        pltpu.sync_copy(z_hbm, accum.at[pl.ds(r0, rows_per_tile)])
        plsc.subcore_barrier()

        def body(ci, carry):
            base = base_w + ci * CH
            pltpu.sync_copy(src_hbm.at[pl.ds(base, CH)], idx_v)
            pltpu.sync_copy(o_hbm.at[pl.ds(base, CH)], rows_v)
            pltpu.sync_copy(rows_v, accum.at[idx_v], add=True)
            return carry

        lax.fori_loop(0, nfull, body, 0)

        bt = base_w + nfull * CH
        pltpu.sync_copy(src_hbm.at[pl.ds(bt, tail)], idx_t)
        pltpu.sync_copy(o_hbm.at[pl.ds(bt, tail)], rows_t)
        pltpu.sync_copy(rows_t, accum.at[idx_t], add=True)

        plsc.subcore_barrier()
        pltpu.sync_copy(accum.at[pl.ds(r0, rows_per_tile)],
                        out_hbm.at[cid, pl.ds(r0, rows_per_tile)])

    return k(o_rows, src, zeros_init)


def _node_body(f_ref, s_ref, p_ref, c_ref, wa_ref, wb_ref, w1f_ref, ws1_ref,
               ws2_ref, b1_ref, w2_ref, b2_ref, w3_ref, b3_ref,
               rep_ref, til_ref, fo_ref, so_ref):
    fb = f_ref[...]
    sb = s_ref[...]
    ps = p_ref[0] + p_ref[1]
    cnt = jnp.maximum(c_ref[0, :, :1] + c_ref[1, :, :1], 1.0)
    inv = 1.0 / cnt
    f_c = ps[:, :96] * inv
    s_c = ps[:, 96:160] * inv

    tf = (jnp.dot(fb, wa_ref[...], preferred_element_type=jnp.float32)
          + jnp.dot(f_c, wb_ref[...], preferred_element_type=jnp.float32))
    sp = (jnp.dot(sb, ws1_ref[...], preferred_element_type=jnp.float32)
          + jnp.dot(s_c, ws2_ref[...], preferred_element_type=jnp.float32))
    f_msg, s_msg = _edge_core(
        tf[:, 0:32], tf[:, 32:64], tf[:, 64:96], sp,
        w1f_ref[...], b1_ref[...], w2_ref[...], b2_ref[...],
        w3_ref[...], b3_ref[...], rep_ref[...], til_ref[...])
    fo_ref[...] = f_msg + fb
    so_ref[...] = s_msg + sb


def _run_node(f_flat, s, partials, counts, wa2, wb2, sw1f, sw1s, sw1sc, sb1,
              sw2, sb2, sw3, sb3, rep_m, til_m):
    nb = N // BN
    full = lambda shp: pl.BlockSpec(shp, lambda i: tuple(0 for _ in shp))
    return pl.pallas_call(
        _node_body,
        grid=(nb,),
        in_specs=[
            pl.BlockSpec((BN, 96), lambda i: (i, 0)),
            pl.BlockSpec((BN, HD), lambda i: (i, 0)),
            pl.BlockSpec((NC, BN, DO), lambda i: (0, i, 0)),
            pl.BlockSpec((NC, BN, DC), lambda i: (0, i, 0)),
            full((96, 96)), full((96, 96)),
            full((1024, HD)), full((HD, HD)), full((HD, HD)), full((1, HD)),
            full((HD, HD)), full((1, HD)), full((HD, 1088)), full((1, 1088)),
            full((VD, 1024)), full((VD, 1024)),
        ],
        out_specs=[
            pl.BlockSpec((BN, 96), lambda i: (i, 0)),
            pl.BlockSpec((BN, HD), lambda i: (i, 0)),
        ],
        out_shape=[
            jax.ShapeDtypeStruct((N, 96), jnp.float32),
            jax.ShapeDtypeStruct((N, HD), jnp.float32),
        ],
    )(f_flat, s, partials, counts, wa2, wb2, sw1f, sw1s, sw1sc, sb1, sw2,
      sb2, sw3, sb3, rep_m, til_m)


def kernel(f, s, edge_index, W_emb1, W_emb2, nW1, nb1, nW2, nb2, nW3, nb3,
           sW1, sb1, sW2, sb2, sW3, sb3):
    f_flat = f.reshape(N, 96)
    src = edge_index[0].astype(jnp.int32)
    dst = edge_index[1].astype(jnp.int32)

    eye3 = jnp.eye(3, dtype=jnp.float32)
    wa96 = jnp.kron(eye3, W_emb1[:VD])
    wb96 = jnp.kron(eye3, W_emb1[VD:])
    wa2 = jnp.kron(eye3, W_emb2[:VD])
    wb2 = jnp.kron(eye3, W_emb2[VD:])

    u_tab, v_tab = _run_prep(f_flat, s, wa96, wb96,
                             nW1[1024:1088], nW1[1088:1152])
    zeros_c = jnp.zeros((N // NS, DC), jnp.float32)
    ones_rows = jnp.ones((CH, DC), jnp.float32)
    gu, gv, counts = _run_gather(u_tab, v_tab, src, dst, zeros_c, ones_rows)
    rep_m, til_m = _onehot_mats()
    o_rows = _run_edge(gu, gv, nW1[:1024], nb1.reshape(1, HD), nW2,
                       nb2.reshape(1, HD), nW3, nb3.reshape(1, 1088),
                       rep_m, til_m)
    zeros_init = jnp.zeros((N // NS, DO), jnp.float32)
    partials = _run_scatter(o_rows, src, zeros_init)
    f_out, s_out = _run_node(f_flat, s, partials, counts, wa2, wb2,
                             sW1[:1024], sW1[1024:1088], sW1[1088:1152],
                             sb1.reshape(1, HD), sW2, sb2.reshape(1, HD),
                             sW3, sb3.reshape(1, 1088), rep_m, til_m)
    return (f_out.reshape(N, 3, VD), s_out)


# speedup vs baseline: 12.3293x; 1.1222x over previous
import functools

import jax
import jax.numpy as jnp
from jax import lax
from jax.experimental import pallas as pl
from jax.experimental.pallas import tpu as pltpu
from jax.experimental.pallas import tpu_sc as plsc

VD = 32
HD = 64
N = 10000
E = 160000
DU = 160
DO = 160
DC = 16

NC, NS = 2, 16
NW = NC * NS
EPW = E // NW
CH = 128

BE = 800
BN = 1000


def _edge_core(F0, F1, F2, sp, w1f, b1, w2, b2, w3, b3, rep_m):
    Fs = (F0, F1, F2)
    Rs = [jnp.dot(Fk, rep_m, preferred_element_type=jnp.float32) for Fk in Fs]
    Ts = [jnp.concatenate([Fk] * 32, axis=1) for Fk in Fs]
    f2s = Rs[0] * Ts[0] + Rs[1] * Ts[1] + Rs[2] * Ts[2]
    nrm2 = jnp.sum(f2s * f2s, axis=1, keepdims=True)
    fnorm = jnp.sqrt(nrm2) + 1.0

    h = jnp.dot(f2s, w1f, preferred_element_type=jnp.float32) + sp + b1
    h = jnp.maximum(h, 0.0)
    h = jnp.dot(h, w2, preferred_element_type=jnp.float32) + b2
    h = jnp.maximum(h, 0.0)
    cvec = jnp.dot(h, w3, preferred_element_type=jnp.float32) + b3
    cvec = cvec * (1.0 / fnorm)

    cmain = cvec[:, :1024]
    outs = []
    for k in range(3):
        p = Rs[k] * cmain
        p = p[:, :512] + p[:, 512:]
        p = p[:, :256] + p[:, 256:]
        p = p[:, :128] + p[:, 128:]
        outs.append(p[:, 0:32] + p[:, 32:64] + p[:, 64:96] + p[:, 96:128])
    f_msg = jnp.concatenate(outs, axis=1)
    s_msg = cvec[:, 1024:1088]
    return f_msg, s_msg


def _onehot_mats():
    col = jnp.arange(1024, dtype=jnp.int32)
    row = jnp.arange(32, dtype=jnp.int32)[:, None]
    rep_m = (col[None, :] // 32 == row).astype(jnp.float32)
    return rep_m


def _prep_body(f_ref, s_ref, wa_ref, wb_ref, ws1_ref, ws2_ref, u_ref, v_ref):
    fb = f_ref[...]
    sb = s_ref[...]
    a = jnp.dot(fb, wa_ref[...], preferred_element_type=jnp.float32)
    b = jnp.dot(fb, wb_ref[...], preferred_element_type=jnp.float32)
    ps = jnp.dot(sb, ws1_ref[...], preferred_element_type=jnp.float32)
    qs = jnp.dot(sb, ws2_ref[...], preferred_element_type=jnp.float32)
    u_ref[...] = jnp.concatenate([a, ps], axis=1)
    v_ref[...] = jnp.concatenate([b, qs], axis=1)


def _run_prep(f_flat, s, wa96, wb96, w1s1, w1s2):
    nb = N // BN
    full = lambda shp: pl.BlockSpec(shp, lambda i: (0, 0))
    return pl.pallas_call(
        _prep_body,
        grid=(nb,),
        in_specs=[
            pl.BlockSpec((BN, 96), lambda i: (i, 0)),
            pl.BlockSpec((BN, HD), lambda i: (i, 0)),
            full((96, 96)), full((96, 96)), full((HD, HD)), full((HD, HD)),
        ],
        out_specs=[
            pl.BlockSpec((BN, DU), lambda i: (i, 0)),
            pl.BlockSpec((BN, DU), lambda i: (i, 0)),
        ],
        out_shape=[
            jax.ShapeDtypeStruct((N, DU), jnp.float32),
            jax.ShapeDtypeStruct((N, DU), jnp.float32),
        ],
    )(f_flat, s, wa96, wb96, w1s1, w1s2)


def _run_gather(u_tab, v_tab, src, dst, zeros_c, ones_rows):
    mesh = plsc.VectorSubcoreMesh(core_axis_name="c", subcore_axis_name="s")
    nfull = EPW // CH
    tail = EPW - nfull * CH
    rows_per_tile = N // NS

    @functools.partial(
        pl.kernel,
        mesh=mesh,
        compiler_params=pltpu.CompilerParams(use_tc_tiling_on_sc=False),
        out_type=[
            jax.ShapeDtypeStruct((E, DU), jnp.float32),
            jax.ShapeDtypeStruct((E, DU), jnp.float32),
            jax.ShapeDtypeStruct((NC, N, DC), jnp.float32),
        ],
        scratch_types=[
            pltpu.VMEM((CH,), jnp.int32),
            pltpu.VMEM((CH, DU), jnp.float32),
            pltpu.VMEM((CH, DC), jnp.float32),
            pltpu.VMEM((tail,), jnp.int32),
            pltpu.VMEM((tail, DU), jnp.float32),
            pltpu.VMEM((tail, DC), jnp.float32),
            pltpu.VMEM_SHARED((N, DC), jnp.float32),
            pltpu.SemaphoreType.DMA,
        ],
    )
    def k(u_hbm, v_hbm, src_hbm, dst_hbm, zc_hbm, ones_hbm,
          gu_hbm, gv_hbm, cnt_hbm,
          idx_v, rows_v, ones_v, idx_t, rows_t, ones_t, cacc, sem):
        cid = lax.axis_index("c")
        sid = lax.axis_index("s")
        base_w = cid * (E // NC) + sid * EPW
        r0 = sid * rows_per_tile

        pltpu.sync_copy(ones_hbm, ones_v)
        pltpu.sync_copy(ones_hbm.at[pl.ds(0, tail)], ones_t)
        pltpu.sync_copy(zc_hbm, cacc.at[pl.ds(r0, rows_per_tile)])
        plsc.subcore_barrier()

        def body(ci, carry):
            base = base_w + ci * CH
            pltpu.sync_copy(src_hbm.at[pl.ds(base, CH)], idx_v)
            pltpu.async_copy(u_hbm.at[idx_v], rows_v, sem).wait()
            pltpu.sync_copy(rows_v, gu_hbm.at[pl.ds(base, CH)])
            pltpu.sync_copy(ones_v, cacc.at[idx_v], add=True)
            pltpu.sync_copy(dst_hbm.at[pl.ds(base, CH)], idx_v)
            pltpu.async_copy(v_hbm.at[idx_v], rows_v, sem).wait()
            pltpu.sync_copy(rows_v, gv_hbm.at[pl.ds(base, CH)])
            return carry

        lax.fori_loop(0, nfull, body, 0)

        bt = base_w + nfull * CH
        pltpu.sync_copy(src_hbm.at[pl.ds(bt, tail)], idx_t)
        pltpu.async_copy(u_hbm.at[idx_t], rows_t, sem).wait()
        pltpu.sync_copy(rows_t, gu_hbm.at[pl.ds(bt, tail)])
        pltpu.sync_copy(ones_t, cacc.at[idx_t], add=True)
        pltpu.sync_copy(dst_hbm.at[pl.ds(bt, tail)], idx_t)
        pltpu.async_copy(v_hbm.at[idx_t], rows_t, sem).wait()
        pltpu.sync_copy(rows_t, gv_hbm.at[pl.ds(bt, tail)])

        plsc.subcore_barrier()
        pltpu.sync_copy(cacc.at[pl.ds(r0, rows_per_tile)],
                        cnt_hbm.at[cid, pl.ds(r0, rows_per_tile)])

    return k(u_tab, v_tab, src, dst, zeros_c, ones_rows)


def _edge_body(gu_ref, gv_ref, w1f_ref, b1_ref, w2_ref, b2_ref, w3_ref,
               b3_ref, rep_ref, o_ref):
    u = gu_ref[...]
    v = gv_ref[...]
    fvec = u[:, :96] + v[:, :96]
    sp = u[:, 96:] + v[:, 96:]
    f_msg, s_msg = _edge_core(
        fvec[:, 0:32], fvec[:, 32:64], fvec[:, 64:96], sp,
        w1f_ref[...], b1_ref[...], w2_ref[...], b2_ref[...],
        w3_ref[...], b3_ref[...], rep_ref[...])
    o_ref[...] = jnp.concatenate([f_msg, s_msg], axis=1)


def _run_edge(gu, gv, w1f, b1, w2, b2, w3, b3, rep_m):
    nb = E // BE
    full = lambda shp: pl.BlockSpec(shp, lambda i: tuple(0 for _ in shp))
    return pl.pallas_call(
        _edge_body,
        grid=(nb,),
        in_specs=[
            pl.BlockSpec((BE, DU), lambda i: (i, 0)),
            pl.BlockSpec((BE, DU), lambda i: (i, 0)),
            full((1024, HD)), full((1, HD)), full((HD, HD)), full((1, HD)),
            full((HD, 1088)), full((1, 1088)),
            full((VD, 1024)),
        ],
        out_specs=pl.BlockSpec((BE, DO), lambda i: (i, 0)),
        out_shape=jax.ShapeDtypeStruct((E, DO), jnp.float32),
    )(gu, gv, w1f, b1, w2, b2, w3, b3, rep_m)


def _run_scatter(o_rows, src, zeros_init):
    mesh = plsc.VectorSubcoreMesh(core_axis_name="c", subcore_axis_name="s")
    nfull = EPW // CH
    tail = EPW - nfull * CH
    rows_per_tile = N // NS

    @functools.partial(
        pl.kernel,
        mesh=mesh,
        compiler_params=pltpu.CompilerParams(use_tc_tiling_on_sc=False),
        out_type=jax.ShapeDtypeStruct((NC, N, DO), jnp.float32),
        scratch_types=[
            pltpu.VMEM((CH,), jnp.int32),
            pltpu.VMEM((CH, DO), jnp.float32),
            pltpu.VMEM((tail,), jnp.int32),
            pltpu.VMEM((tail, DO), jnp.float32),
            pltpu.VMEM_SHARED((N, DO), jnp.float32),
        ],
    )
    def k(o_hbm, src_hbm, z_hbm, out_hbm, idx_v, rows_v, idx_t, rows_t, accum):
        cid = lax.axis_index("c")
        sid = lax.axis_index("s")
        base_w = cid * (E // NC) + sid * EPW
        r0 = sid * rows_per_tile

        pltpu.sync_copy(z_hbm, accum.at[pl.ds(r0, rows_per_tile)])
        plsc.subcore_barrier()

        def body(ci, carry):
            base = base_w + ci * CH
            pltpu.sync_copy(src_hbm.at[pl.ds(base, CH)], idx_v)
            pltpu.sync_copy(o_hbm.at[pl.ds(base, CH)], rows_v)
            pltpu.sync_copy(rows_v, accum.at[idx_v], add=True)
            return carry

        lax.fori_loop(0, nfull, body, 0)

        bt = base_w + nfull * CH
        pltpu.sync_copy(src_hbm.at[pl.ds(bt, tail)], idx_t)
        pltpu.sync_copy(o_hbm.at[pl.ds(bt, tail)], rows_t)
        pltpu.sync_copy(rows_t, accum.at[idx_t], add=True)

        plsc.subcore_barrier()
        pltpu.sync_copy(accum.at[pl.ds(r0, rows_per_tile)],
                        out_hbm.at[cid, pl.ds(r0, rows_per_tile)])

    return k(o_rows, src, zeros_init)


def _node_body(f_ref, s_ref, p_ref, c_ref, wa_ref, wb_ref, w1f_ref, ws1_ref,
               ws2_ref, b1_ref, w2_ref, b2_ref, w3_ref, b3_ref,
               rep_ref, fo_ref, so_ref):
    fb = f_ref[...]
    sb = s_ref[...]
    ps = p_ref[0] + p_ref[1]
    cnt = jnp.maximum(c_ref[0, :, :1] + c_ref[1, :, :1], 1.0)
    inv = 1.0 / cnt
    f_c = ps[:, :96] * inv
    s_c = ps[:, 96:160] * inv

    tf = (jnp.dot(fb, wa_ref[...], preferred_element_type=jnp.float32)
          + jnp.dot(f_c, wb_ref[...], preferred_element_type=jnp.float32))
    sp = (jnp.dot(sb, ws1_ref[...], preferred_element_type=jnp.float32)
          + jnp.dot(s_c, ws2_ref[...], preferred_element_type=jnp.float32))
    f_msg, s_msg = _edge_core(
        tf[:, 0:32], tf[:, 32:64], tf[:, 64:96], sp,
        w1f_ref[...], b1_ref[...], w2_ref[...], b2_ref[...],
        w3_ref[...], b3_ref[...], rep_ref[...])
    fo_ref[...] = f_msg + fb
    so_ref[...] = s_msg + sb


def _run_node(f_flat, s, partials, counts, wa2, wb2, sw1f, sw1s, sw1sc, sb1,
              sw2, sb2, sw3, sb3, rep_m):
    nb = N // BN
    full = lambda shp: pl.BlockSpec(shp, lambda i: tuple(0 for _ in shp))
    return pl.pallas_call(
        _node_body,
        grid=(nb,),
        in_specs=[
            pl.BlockSpec((BN, 96), lambda i: (i, 0)),
            pl.BlockSpec((BN, HD), lambda i: (i, 0)),
            pl.BlockSpec((NC, BN, DO), lambda i: (0, i, 0)),
            pl.BlockSpec((NC, BN, DC), lambda i: (0, i, 0)),
            full((96, 96)), full((96, 96)),
            full((1024, HD)), full((HD, HD)), full((HD, HD)), full((1, HD)),
            full((HD, HD)), full((1, HD)), full((HD, 1088)), full((1, 1088)),
            full((VD, 1024)),
        ],
        out_specs=[
            pl.BlockSpec((BN, 96), lambda i: (i, 0)),
            pl.BlockSpec((BN, HD), lambda i: (i, 0)),
        ],
        out_shape=[
            jax.ShapeDtypeStruct((N, 96), jnp.float32),
            jax.ShapeDtypeStruct((N, HD), jnp.float32),
        ],
    )(f_flat, s, partials, counts, wa2, wb2, sw1f, sw1s, sw1sc, sb1, sw2,
      sb2, sw3, sb3, rep_m)


def kernel(f, s, edge_index, W_emb1, W_emb2, nW1, nb1, nW2, nb2, nW3, nb3,
           sW1, sb1, sW2, sb2, sW3, sb3):
    f_flat = f.reshape(N, 96)
    src = edge_index[0].astype(jnp.int32)
    dst = edge_index[1].astype(jnp.int32)

    eye3 = jnp.eye(3, dtype=jnp.float32)
    wa96 = jnp.kron(eye3, W_emb1[:VD])
    wb96 = jnp.kron(eye3, W_emb1[VD:])
    wa2 = jnp.kron(eye3, W_emb2[:VD])
    wb2 = jnp.kron(eye3, W_emb2[VD:])

    u_tab, v_tab = _run_prep(f_flat, s, wa96, wb96,
                             nW1[1024:1088], nW1[1088:1152])
    zeros_c = jnp.zeros((N // NS, DC), jnp.float32)
    ones_rows = jnp.ones((CH, DC), jnp.float32)
    gu, gv, counts = _run_gather(u_tab, v_tab, src, dst, zeros_c, ones_rows)
    rep_m = _onehot_mats()
    o_rows = _run_edge(gu, gv, nW1[:1024], nb1.reshape(1, HD), nW2,
                       nb2.reshape(1, HD), nW3, nb3.reshape(1, 1088),
                       rep_m)
    zeros_init = jnp.zeros((N // NS, DO), jnp.float32)
    partials = _run_scatter(o_rows, src, zeros_init)
    f_out, s_out = _run_node(f_flat, s, partials, counts, wa2, wb2,
                             sW1[:1024], sW1[1024:1088], sW1[1088:1152],
                             sb1.reshape(1, HD), sW2, sb2.reshape(1, HD),
                             sW3, sb3.reshape(1, 1088), rep_m)
    return (f_out.reshape(N, 3, VD), s_out)


# speedup vs baseline: 15.8396x; 1.2847x over previous
import functools

import jax
import jax.numpy as jnp
from jax import lax
from jax.experimental import pallas as pl
from jax.experimental.pallas import tpu as pltpu
from jax.experimental.pallas import tpu_sc as plsc

VD = 32
HD = 64
N = 10000
E = 160000
DU = 256
DO = 256

DU_USED = 160
NC, NS = 2, 16
NW = NC * NS
EPW = E // NW
EPT = E // NS
CH = 128

BE = 800
BN = 1000


def _edge_core(F0, F1, F2, sp, w1f, b1, w2, b2, w3, b3, rep_m):
    Fs = (F0, F1, F2)
    Rs = [jnp.dot(Fk, rep_m, preferred_element_type=jnp.float32) for Fk in Fs]
    Ts = [jnp.concatenate([Fk] * 32, axis=1) for Fk in Fs]
    f2s = Rs[0] * Ts[0] + Rs[1] * Ts[1] + Rs[2] * Ts[2]
    nrm2 = jnp.sum(f2s * f2s, axis=1, keepdims=True)
    fnorm = jnp.sqrt(nrm2) + 1.0

    h = jnp.dot(f2s, w1f, preferred_element_type=jnp.float32) + sp + b1
    h = jnp.maximum(h, 0.0)
    h = jnp.dot(h, w2, preferred_element_type=jnp.float32) + b2
    h = jnp.maximum(h, 0.0)
    cvec = jnp.dot(h, w3, preferred_element_type=jnp.float32) + b3
    cvec = cvec * (1.0 / fnorm)

    cmain = cvec[:, :1024]
    outs = []
    for k in range(3):
        p = Rs[k] * cmain
        p = p[:, :512] + p[:, 512:]
        p = p[:, :256] + p[:, 256:]
        p = p[:, :128] + p[:, 128:]
        outs.append(p[:, 0:32] + p[:, 32:64] + p[:, 64:96] + p[:, 96:128])
    f_msg = jnp.concatenate(outs, axis=1)
    s_msg = cvec[:, 1024:1088]
    return f_msg, s_msg


def _onehot_mats():
    col = jnp.arange(1024, dtype=jnp.int32)
    row = jnp.arange(32, dtype=jnp.int32)[:, None]
    rep_m = (col[None, :] // 32 == row).astype(jnp.float32)
    return rep_m


def _prep_body(f_ref, s_ref, wa_ref, wb_ref, ws1_ref, ws2_ref, u_ref, v_ref):
    fb = f_ref[...]
    sb = s_ref[...]
    a = jnp.dot(fb, wa_ref[...], preferred_element_type=jnp.float32)
    b = jnp.dot(fb, wb_ref[...], preferred_element_type=jnp.float32)
    ps = jnp.dot(sb, ws1_ref[...], preferred_element_type=jnp.float32)
    qs = jnp.dot(sb, ws2_ref[...], preferred_element_type=jnp.float32)
    z = jnp.zeros((fb.shape[0], DU - DU_USED), jnp.float32)
    u_ref[...] = jnp.concatenate([a, ps, z], axis=1)
    v_ref[...] = jnp.concatenate([b, qs, z], axis=1)


def _run_prep(f_flat, s, wa96, wb96, w1s1, w1s2):
    nb = N // BN
    full = lambda shp: pl.BlockSpec(shp, lambda i: (0, 0))
    return pl.pallas_call(
        _prep_body,
        grid=(nb,),
        in_specs=[
            pl.BlockSpec((BN, 96), lambda i: (i, 0)),
            pl.BlockSpec((BN, HD), lambda i: (i, 0)),
            full((96, 96)), full((96, 96)), full((HD, HD)), full((HD, HD)),
        ],
        out_specs=[
            pl.BlockSpec((BN, DU), lambda i: (i, 0)),
            pl.BlockSpec((BN, DU), lambda i: (i, 0)),
        ],
        out_shape=[
            jax.ShapeDtypeStruct((N, DU), jnp.float32),
            jax.ShapeDtypeStruct((N, DU), jnp.float32),
        ],
    )(f_flat, s, wa96, wb96, w1s1, w1s2)


def _run_gather(u_tab, v_tab, src, dst):
    mesh = plsc.VectorSubcoreMesh(core_axis_name="c", subcore_axis_name="s")
    nch = (EPW + CH - 1) // CH

    @functools.partial(
        pl.kernel,
        mesh=mesh,
        out_type=[
            jax.ShapeDtypeStruct((E, DU), jnp.float32),
            jax.ShapeDtypeStruct((E, DU), jnp.float32),
        ],
        scratch_types=[
            pltpu.VMEM((CH,), jnp.int32),
            pltpu.VMEM((CH,), jnp.int32),
            pltpu.VMEM((CH, DU), jnp.float32),
            pltpu.VMEM((CH, DU), jnp.float32),
            pltpu.SemaphoreType.DMA,
            pltpu.SemaphoreType.DMA,
            pltpu.SemaphoreType.DMA,
            pltpu.SemaphoreType.DMA,
        ],
    )
    def k(u_hbm, v_hbm, src_hbm, dst_hbm, gu_hbm, gv_hbm,
          idx_u, idx_d, urows, vrows, gsem_u, gsem_v, ssem_u, ssem_v):
        cid = lax.axis_index("c")
        sid = lax.axis_index("s")
        base_w = cid * (E // NC) + sid * EPW

        def body(ci, carry):
            base = pl.multiple_of(jnp.minimum(base_w + ci * CH, E - CH), 8)
            pltpu.sync_copy(src_hbm.at[pl.ds(base, CH)], idx_u)
            gu = pltpu.async_copy(u_hbm.at[idx_u], urows, gsem_u)
            pltpu.sync_copy(dst_hbm.at[pl.ds(base, CH)], idx_d)
            gv = pltpu.async_copy(v_hbm.at[idx_d], vrows, gsem_v)
            gu.wait()
            su = pltpu.async_copy(urows, gu_hbm.at[pl.ds(base, CH)], ssem_u)
            gv.wait()
            sv = pltpu.async_copy(vrows, gv_hbm.at[pl.ds(base, CH)], ssem_v)
            su.wait()
            sv.wait()
            return carry

        lax.fori_loop(0, nch, body, 0)

    return k(u_tab, v_tab, src, dst)


def _edge_body(gu_ref, gv_ref, w1f_ref, b1_ref, w2_ref, b2_ref, w3_ref,
               b3_ref, rep_ref, o_ref):
    u = gu_ref[...]
    v = gv_ref[...]
    fvec = u[:, :96] + v[:, :96]
    sp = u[:, 96:DU_USED] + v[:, 96:DU_USED]
    f_msg, s_msg = _edge_core(
        fvec[:, 0:32], fvec[:, 32:64], fvec[:, 64:96], sp,
        w1f_ref[...], b1_ref[...], w2_ref[...], b2_ref[...],
        w3_ref[...], b3_ref[...], rep_ref[...])
    B = f_msg.shape[0]
    ones = jnp.ones((B, 16), jnp.float32)
    zpad = jnp.zeros((B, DO - DU_USED - 16), jnp.float32)
    o_ref[...] = jnp.concatenate([f_msg, s_msg, ones, zpad], axis=1)


def _run_edge(gu, gv, w1f, b1, w2, b2, w3, b3, rep_m):
    nb = E // BE
    full = lambda shp: pl.BlockSpec(shp, lambda i: tuple(0 for _ in shp))
    return pl.pallas_call(
        _edge_body,
        grid=(nb,),
        in_specs=[
            pl.BlockSpec((BE, DU), lambda i: (i, 0)),
            pl.BlockSpec((BE, DU), lambda i: (i, 0)),
            full((1024, HD)), full((1, HD)), full((HD, HD)), full((1, HD)),
            full((HD, 1088)), full((1, 1088)),
            full((VD, 1024)),
        ],
        out_specs=pl.BlockSpec((BE, DO), lambda i: (i, 0)),
        out_shape=jax.ShapeDtypeStruct((E, DO), jnp.float32),
    )(gu, gv, w1f, b1, w2, b2, w3, b3, rep_m)


HW = DO // NC
N_ACC = 10240


def _run_scatter(o_rows, src, zeros_init):
    mesh = plsc.VectorSubcoreMesh(core_axis_name="c", subcore_axis_name="s")
    nfull = EPT // CH
    tail = EPT - nfull * CH
    rows_per_tile = N_ACC // NS

    @functools.partial(
        pl.kernel,
        mesh=mesh,
        out_type=jax.ShapeDtypeStruct((N_ACC, DO), jnp.float32),
        scratch_types=[
            pltpu.VMEM((CH,), jnp.int32),
            pltpu.VMEM((CH, HW), jnp.float32),
            pltpu.VMEM((tail,), jnp.int32),
            pltpu.VMEM((tail, HW), jnp.float32),
            pltpu.VMEM_SHARED((N_ACC, HW), jnp.float32),
        ],
    )
    def k(o_hbm, src_hbm, z_hbm, out_hbm, idx_v, rows_v, idx_t, rows_t, accum):
        cid = lax.axis_index("c")
        sid = lax.axis_index("s")
        base_w = sid * EPT
        coff = pl.multiple_of(cid * HW, HW)
        r0 = pl.multiple_of(sid * rows_per_tile, 8)

        pltpu.sync_copy(z_hbm, accum.at[pl.ds(r0, rows_per_tile)])
        plsc.subcore_barrier()

        def body(ci, carry):
            base = pl.multiple_of(base_w + ci * CH, 8)
            pltpu.sync_copy(src_hbm.at[pl.ds(base, CH)], idx_v)
            pltpu.sync_copy(o_hbm.at[pl.ds(base, CH), pl.ds(coff, HW)],
                            rows_v)
            pltpu.sync_copy(rows_v, accum.at[idx_v], add=True)
            return carry

        lax.fori_loop(0, nfull, body, 0)

        bt = pl.multiple_of(base_w + nfull * CH, 8)
        pltpu.sync_copy(src_hbm.at[pl.ds(bt, tail)], idx_t)
        pltpu.sync_copy(o_hbm.at[pl.ds(bt, tail), pl.ds(coff, HW)], rows_t)
        pltpu.sync_copy(rows_t, accum.at[idx_t], add=True)

        plsc.subcore_barrier()
        pltpu.sync_copy(accum.at[pl.ds(r0, rows_per_tile)],
                        out_hbm.at[pl.ds(r0, rows_per_tile),
                                   pl.ds(coff, HW)])

    return k(o_rows, src, zeros_init)


def _node_body(f_ref, s_ref, p_ref, wa_ref, wb_ref, w1f_ref, ws1_ref,
               ws2_ref, b1_ref, w2_ref, b2_ref, w3_ref, b3_ref,
               rep_ref, fo_ref, so_ref):
    fb = f_ref[...]
    sb = s_ref[...]
    ps = p_ref[...]
    cnt = jnp.maximum(ps[:, 160:161], 1.0)
    inv = 1.0 / cnt
    f_c = ps[:, :96] * inv
    s_c = ps[:, 96:160] * inv

    tf = (jnp.dot(fb, wa_ref[...], preferred_element_type=jnp.float32)
          + jnp.dot(f_c, wb_ref[...], preferred_element_type=jnp.float32))
    sp = (jnp.dot(sb, ws1_ref[...], preferred_element_type=jnp.float32)
          + jnp.dot(s_c, ws2_ref[...], preferred_element_type=jnp.float32))
    f_msg, s_msg = _edge_core(
        tf[:, 0:32], tf[:, 32:64], tf[:, 64:96], sp,
        w1f_ref[...], b1_ref[...], w2_ref[...], b2_ref[...],
        w3_ref[...], b3_ref[...], rep_ref[...])
    fo_ref[...] = f_msg + fb
    so_ref[...] = s_msg + sb


def _run_node(f_flat, s, partials, wa2, wb2, sw1f, sw1s, sw1sc, sb1,
              sw2, sb2, sw3, sb3, rep_m):
    nb = N // BN
    full = lambda shp: pl.BlockSpec(shp, lambda i: tuple(0 for _ in shp))
    return pl.pallas_call(
        _node_body,
        grid=(nb,),
        in_specs=[
            pl.BlockSpec((BN, 96), lambda i: (i, 0)),
            pl.BlockSpec((BN, HD), lambda i: (i, 0)),
            pl.BlockSpec((BN, DO), lambda i: (i, 0)),
            full((96, 96)), full((96, 96)),
            full((1024, HD)), full((HD, HD)), full((HD, HD)), full((1, HD)),
            full((HD, HD)), full((1, HD)), full((HD, 1088)), full((1, 1088)),
            full((VD, 1024)),
        ],
        out_specs=[
            pl.BlockSpec((BN, 96), lambda i: (i, 0)),
            pl.BlockSpec((BN, HD), lambda i: (i, 0)),
        ],
        out_shape=[
            jax.ShapeDtypeStruct((N, 96), jnp.float32),
            jax.ShapeDtypeStruct((N, HD), jnp.float32),
        ],
    )(f_flat, s, partials, wa2, wb2, sw1f, sw1s, sw1sc, sb1, sw2,
      sb2, sw3, sb3, rep_m)


def kernel(f, s, edge_index, W_emb1, W_emb2, nW1, nb1, nW2, nb2, nW3, nb3,
           sW1, sb1, sW2, sb2, sW3, sb3):
    f_flat = f.reshape(N, 96)
    src = edge_index[0].astype(jnp.int32)
    dst = edge_index[1].astype(jnp.int32)

    eye3 = jnp.eye(3, dtype=jnp.float32)
    wa96 = jnp.kron(eye3, W_emb1[:VD])
    wb96 = jnp.kron(eye3, W_emb1[VD:])
    wa2 = jnp.kron(eye3, W_emb2[:VD])
    wb2 = jnp.kron(eye3, W_emb2[VD:])

    u_tab, v_tab = _run_prep(f_flat, s, wa96, wb96,
                             nW1[1024:1088], nW1[1088:1152])
    gu, gv = _run_gather(u_tab, v_tab, src, dst)
    rep_m = _onehot_mats()
    o_rows = _run_edge(gu, gv, nW1[:1024], nb1.reshape(1, HD), nW2,
                       nb2.reshape(1, HD), nW3, nb3.reshape(1, 1088),
                       rep_m)
    zeros_init = jnp.zeros((N_ACC // NS, HW), jnp.float32)
    partials = _run_scatter(o_rows, src, zeros_init)[:N]
    f_out, s_out = _run_node(f_flat, s, partials, wa2, wb2,
                             sW1[:1024], sW1[1024:1088], sW1[1088:1152],
                             sb1.reshape(1, HD), sW2, sb2.reshape(1, HD),
                             sW3, sb3.reshape(1, 1088), rep_m)
    return (f_out.reshape(N, 3, VD), s_out)


# speedup vs baseline: 16.9838x; 1.0722x over previous
import functools

import jax
import jax.numpy as jnp
from jax import lax
from jax.experimental import pallas as pl
from jax.experimental.pallas import tpu as pltpu
from jax.experimental.pallas import tpu_sc as plsc

VD = 32
HD = 64
N = 10000
E = 160000
DIN = VD * VD + 2 * HD
DU = 256
DO = 256

DU_USED = 160
NC, NS = 2, 16
NW = NC * NS
EPW = E // NW
EPT = E // NS
CH = 128

BE = 800
BN = 1000


def _edge_core(F0, F1, F2, sp, w1f, b1, w2, b2, w3, b3, rep_m):
    Fs = (F0, F1, F2)
    Rs = [jnp.dot(Fk, rep_m, preferred_element_type=jnp.float32) for Fk in Fs]
    Ts = [jnp.concatenate([Fk] * 32, axis=1) for Fk in Fs]
    f2s = Rs[0] * Ts[0] + Rs[1] * Ts[1] + Rs[2] * Ts[2]
    nrm2 = jnp.sum(f2s * f2s, axis=1, keepdims=True)
    fnorm = jnp.sqrt(nrm2) + 1.0

    h = jnp.dot(f2s, w1f, preferred_element_type=jnp.float32) + sp + b1
    h = jnp.maximum(h, 0.0)
    h = jnp.dot(h, w2, preferred_element_type=jnp.float32) + b2
    h = jnp.maximum(h, 0.0)
    cvec = jnp.dot(h, w3, preferred_element_type=jnp.float32) + b3
    cvec = cvec * (1.0 / fnorm)

    cmain = cvec[:, :1024]
    outs = []
    for k in range(3):
        p = Rs[k] * cmain
        p = p[:, :512] + p[:, 512:]
        p = p[:, :256] + p[:, 256:]
        p = p[:, :128] + p[:, 128:]
        outs.append(p[:, 0:32] + p[:, 32:64] + p[:, 64:96] + p[:, 96:128])
    f_msg = jnp.concatenate(outs, axis=1)
    s_msg = cvec[:, 1024:1088]
    return f_msg, s_msg


def _onehot_mats():
    col = jnp.arange(1024, dtype=jnp.int32)
    row = jnp.arange(32, dtype=jnp.int32)[:, None]
    rep_m = (col[None, :] // 32 == row).astype(jnp.float32)
    return rep_m


def _blockdiag_mm(fb, w):
    return jnp.concatenate(
        [jnp.dot(fb[:, 32 * k:32 * (k + 1)], w,
                 preferred_element_type=jnp.float32) for k in range(3)],
        axis=1)


def _prep_body(f_ref, s_ref, we1_ref, w1_ref, u_ref, v_ref):
    fb = f_ref[...]
    sb = s_ref[...]
    we1 = we1_ref[...]
    a = _blockdiag_mm(fb, we1[:VD])
    b = _blockdiag_mm(fb, we1[VD:])
    ps = jnp.dot(sb, w1_ref[1024:1088], preferred_element_type=jnp.float32)
    qs = jnp.dot(sb, w1_ref[1088:1152], preferred_element_type=jnp.float32)
    z = jnp.zeros((fb.shape[0], DU - DU_USED), jnp.float32)
    u_ref[...] = jnp.concatenate([a, ps, z], axis=1)
    v_ref[...] = jnp.concatenate([b, qs, z], axis=1)


def _run_prep(f_flat, s, w_emb1, nw1):
    nb = N // BN
    full = lambda shp: pl.BlockSpec(shp, lambda i: (0, 0))
    return pl.pallas_call(
        _prep_body,
        grid=(nb,),
        in_specs=[
            pl.BlockSpec((BN, 96), lambda i: (i, 0)),
            pl.BlockSpec((BN, HD), lambda i: (i, 0)),
            full((2 * VD, VD)), full((DIN, HD)),
        ],
        out_specs=[
            pl.BlockSpec((BN, DU), lambda i: (i, 0)),
            pl.BlockSpec((BN, DU), lambda i: (i, 0)),
        ],
        out_shape=[
            jax.ShapeDtypeStruct((N, DU), jnp.float32),
            jax.ShapeDtypeStruct((N, DU), jnp.float32),
        ],
    )(f_flat, s, w_emb1, nw1)


G_STRIDE = 4992
G_NCH = 41


def _run_gather(u_tab, v_tab, src, dst):
    mesh = plsc.VectorSubcoreMesh(core_axis_name="c", subcore_axis_name="s")
    nidx = G_NCH * CH

    @functools.partial(
        pl.kernel,
        mesh=mesh,
        out_type=[
            jax.ShapeDtypeStruct((E, DU), jnp.float32),
            jax.ShapeDtypeStruct((E, DU), jnp.float32),
        ],
        scratch_types=[
            pltpu.VMEM((nidx,), jnp.int32),
            pltpu.VMEM((nidx,), jnp.int32),
            pltpu.VMEM((CH, DU), jnp.float32),
            pltpu.VMEM((CH, DU), jnp.float32),
            pltpu.SemaphoreType.DMA,
            pltpu.SemaphoreType.DMA,
            pltpu.SemaphoreType.DMA,
            pltpu.SemaphoreType.DMA,
        ],
    )
    def k(u_hbm, v_hbm, src_hbm, dst_hbm, gu_hbm, gv_hbm,
          idx_u, idx_d, urows, vrows, gsem_u, gsem_v, ssem_u, ssem_v):
        cid = lax.axis_index("c")
        sid = lax.axis_index("s")
        w = sid * NC + cid
        base_w = pl.multiple_of(w * G_STRIDE, 8)

        pltpu.sync_copy(src_hbm.at[pl.ds(base_w, nidx)], idx_u)
        pltpu.sync_copy(dst_hbm.at[pl.ds(base_w, nidx)], idx_d)

        def start(ci):
            lo = pl.multiple_of(ci * CH, 8)
            pltpu.async_copy(u_hbm.at[idx_u.at[pl.ds(lo, CH)]],
                             urows, gsem_u)
            pltpu.async_copy(v_hbm.at[idx_d.at[pl.ds(lo, CH)]],
                             vrows, gsem_v)

        start(0)

        def body(ci, carry):
            base = pl.multiple_of(base_w + ci * CH, 8)
            lo = pl.multiple_of(ci * CH, 8)
            pltpu.make_async_copy(u_hbm.at[idx_u.at[pl.ds(lo, CH)]],
                                  urows, gsem_u).wait()
            su = pltpu.async_copy(urows, gu_hbm.at[pl.ds(base, CH)], ssem_u)
            pltpu.make_async_copy(v_hbm.at[idx_d.at[pl.ds(lo, CH)]],
                                  vrows, gsem_v).wait()
            sv = pltpu.async_copy(vrows, gv_hbm.at[pl.ds(base, CH)], ssem_v)
            su.wait()
            sv.wait()

            @pl.when(ci < G_NCH - 1)
            def _():
                start(ci + 1)

            return carry

        lax.fori_loop(0, G_NCH, body, 0)

    return k(u_tab, v_tab, src, dst)


def _edge_body(gu_ref, gv_ref, w1_ref, b1_ref, w2_ref, b2_ref, w3_ref,
               b3_ref, rep_ref, o_ref):
    u = gu_ref[...]
    v = gv_ref[...]
    fvec = u[:, :96] + v[:, :96]
    sp = u[:, 96:DU_USED] + v[:, 96:DU_USED]
    f_msg, s_msg = _edge_core(
        fvec[:, 0:32], fvec[:, 32:64], fvec[:, 64:96], sp,
        w1_ref[:1024], b1_ref[...], w2_ref[...], b2_ref[...],
        w3_ref[...], b3_ref[...], rep_ref[...])
    B = f_msg.shape[0]
    ones = jnp.ones((B, 16), jnp.float32)
    zpad = jnp.zeros((B, DO - DU_USED - 16), jnp.float32)
    o_ref[...] = jnp.concatenate([f_msg, s_msg, ones, zpad], axis=1)


def _run_edge(gu, gv, w1, b1, w2, b2, w3, b3, rep_m):
    nb = E // BE
    full = lambda shp: pl.BlockSpec(shp, lambda i: tuple(0 for _ in shp))
    return pl.pallas_call(
        _edge_body,
        grid=(nb,),
        in_specs=[
            pl.BlockSpec((BE, DU), lambda i: (i, 0)),
            pl.BlockSpec((BE, DU), lambda i: (i, 0)),
            full((DIN, HD)), full((1, HD)), full((HD, HD)), full((1, HD)),
            full((HD, 1088)), full((1, 1088)),
            full((VD, 1024)),
        ],
        out_specs=pl.BlockSpec((BE, DO), lambda i: (i, 0)),
        out_shape=jax.ShapeDtypeStruct((E, DO), jnp.float32),
    )(gu, gv, w1, b1, w2, b2, w3, b3, rep_m)


HW = DO // NC
N_ACC = 10240


def _run_scatter(o_rows, src, zeros_init):
    mesh = plsc.VectorSubcoreMesh(core_axis_name="c", subcore_axis_name="s")
    nfull = EPT // CH
    tail = EPT - nfull * CH
    rows_per_tile = N_ACC // NS

    @functools.partial(
        pl.kernel,
        mesh=mesh,
        out_type=jax.ShapeDtypeStruct((N_ACC, DO), jnp.float32),
        scratch_types=[
            pltpu.VMEM((CH,), jnp.int32),
            pltpu.VMEM((CH,), jnp.int32),
            pltpu.VMEM((CH, HW), jnp.float32),
            pltpu.VMEM((CH, HW), jnp.float32),
            pltpu.VMEM((tail,), jnp.int32),
            pltpu.VMEM((tail, HW), jnp.float32),
            pltpu.VMEM_SHARED((N_ACC, HW), jnp.float32),
            pltpu.SemaphoreType.DMA,
            pltpu.SemaphoreType.DMA,
            pltpu.SemaphoreType.DMA,
            pltpu.SemaphoreType.DMA,
        ],
    )
    def k(o_hbm, src_hbm, z_hbm, out_hbm, idx0, idx1, rows0, rows1,
          idx_t, rows_t, accum, isem0, isem1, rsem0, rsem1):
        cid = lax.axis_index("c")
        sid = lax.axis_index("s")
        base_w = sid * EPT
        coff = pl.multiple_of(cid * HW, HW)
        r0 = pl.multiple_of(sid * rows_per_tile, 8)
        bufs = ((idx0, rows0, isem0, rsem0), (idx1, rows1, isem1, rsem1))

        pltpu.sync_copy(z_hbm, accum.at[pl.ds(r0, rows_per_tile)])
        plsc.subcore_barrier()

        def start(ci, ib, rb, isem, rsem):
            base = pl.multiple_of(base_w + ci * CH, 8)
            pltpu.async_copy(src_hbm.at[pl.ds(base, CH)], ib, isem)
            pltpu.async_copy(o_hbm.at[pl.ds(base, CH), pl.ds(coff, HW)],
                             rb, rsem)

        for b in range(2):
            start(b, *bufs[b])

        def body(m, carry):
            for b in range(2):
                ci = 2 * m + b
                ib, rb, isem, rsem = bufs[b]
                base = pl.multiple_of(base_w + ci * CH, 8)
                pltpu.make_async_copy(src_hbm.at[pl.ds(base, CH)],
                                      ib, isem).wait()
                pltpu.make_async_copy(
                    o_hbm.at[pl.ds(base, CH), pl.ds(coff, HW)],
                    rb, rsem).wait()
                pltpu.sync_copy(rb, accum.at[ib], add=True)

                @pl.when(ci + 2 < nfull)
                def _():
                    start(ci + 2, ib, rb, isem, rsem)

            return carry

        lax.fori_loop(0, nfull // 2, body, 0)

        bt = pl.multiple_of(base_w + nfull * CH, 8)
        pltpu.sync_copy(src_hbm.at[pl.ds(bt, tail)], idx_t)
        pltpu.sync_copy(o_hbm.at[pl.ds(bt, tail), pl.ds(coff, HW)], rows_t)
        pltpu.sync_copy(rows_t, accum.at[idx_t], add=True)

        plsc.subcore_barrier()
        pltpu.sync_copy(accum.at[pl.ds(r0, rows_per_tile)],
                        out_hbm.at[pl.ds(r0, rows_per_tile),
                                   pl.ds(coff, HW)])

    return k(o_rows, src, zeros_init)


def _node_body(f_ref, s_ref, p_ref, we2_ref, sw1_ref, b1_ref, w2_ref,
               b2_ref, w3_ref, b3_ref, rep_ref, fo_ref, so_ref):
    fb = f_ref[...]
    sb = s_ref[...]
    ps = p_ref[...]
    cnt = jnp.maximum(ps[:, 160:161], 1.0)
    inv = 1.0 / cnt
    f_c = ps[:, :96] * inv
    s_c = ps[:, 96:160] * inv

    we2 = we2_ref[...]
    tf = _blockdiag_mm(fb, we2[:VD])
    tf = tf + _blockdiag_mm(f_c, we2[VD:])
    sp = (jnp.dot(sb, sw1_ref[1024:1088], preferred_element_type=jnp.float32)
          + jnp.dot(s_c, sw1_ref[1088:1152],
                    preferred_element_type=jnp.float32))
    f_msg, s_msg = _edge_core(
        tf[:, 0:32], tf[:, 32:64], tf[:, 64:96], sp,
        sw1_ref[:1024], b1_ref[...], w2_ref[...], b2_ref[...],
        w3_ref[...], b3_ref[...], rep_ref[...])
    fo_ref[...] = f_msg + fb
    so_ref[...] = s_msg + sb


def _run_node(f_flat, s, partials, w_emb2, sw1, sb1, sw2, sb2, sw3, sb3,
              rep_m):
    nb = N // BN
    full = lambda shp: pl.BlockSpec(shp, lambda i: tuple(0 for _ in shp))
    return pl.pallas_call(
        _node_body,
        grid=(nb,),
        in_specs=[
            pl.BlockSpec((BN, 96), lambda i: (i, 0)),
            pl.BlockSpec((BN, HD), lambda i: (i, 0)),
            pl.BlockSpec((BN, DO), lambda i: (i, 0)),
            full((2 * VD, VD)), full((DIN, HD)), full((1, HD)),
            full((HD, HD)), full((1, HD)), full((HD, 1088)), full((1, 1088)),
            full((VD, 1024)),
        ],
        out_specs=[
            pl.BlockSpec((BN, 96), lambda i: (i, 0)),
            pl.BlockSpec((BN, HD), lambda i: (i, 0)),
        ],
        out_shape=[
            jax.ShapeDtypeStruct((N, 96), jnp.float32),
            jax.ShapeDtypeStruct((N, HD), jnp.float32),
        ],
    )(f_flat, s, partials, w_emb2, sw1, sb1, sw2, sb2, sw3, sb3, rep_m)


def kernel(f, s, edge_index, W_emb1, W_emb2, nW1, nb1, nW2, nb2, nW3, nb3,
           sW1, sb1, sW2, sb2, sW3, sb3):
    f_flat = f.reshape(N, 96)
    src = edge_index[0].astype(jnp.int32)
    dst = edge_index[1].astype(jnp.int32)

    u_tab, v_tab = _run_prep(f_flat, s, W_emb1, nW1)
    gu, gv = _run_gather(u_tab, v_tab, src, dst)
    rep_m = _onehot_mats()
    o_rows = _run_edge(gu, gv, nW1, nb1.reshape(1, HD), nW2,
                       nb2.reshape(1, HD), nW3, nb3.reshape(1, 1088),
                       rep_m)
    zeros_init = jnp.zeros((N_ACC // NS, HW), jnp.float32)
    partials = _run_scatter(o_rows, src, zeros_init)[:N]
    f_out, s_out = _run_node(f_flat, s, partials, W_emb2, sW1,
                             sb1.reshape(1, HD), sW2, sb2.reshape(1, HD),
                             sW3, sb3.reshape(1, 1088), rep_m)
    return (f_out.reshape(N, 3, VD), s_out)


# speedup vs baseline: 17.0617x; 1.0046x over previous
import functools

import jax
import jax.numpy as jnp
from jax import lax
from jax.experimental import pallas as pl
from jax.experimental.pallas import tpu as pltpu
from jax.experimental.pallas import tpu_sc as plsc

VD = 32
HD = 64
N = 10000
E = 160000
DIN = VD * VD + 2 * HD
DU = 256
DO = 256

DU_USED = 160
NC, NS = 2, 16
NW = NC * NS
EPW = E // NW
EPT = E // NS
CH = 128

BE = 1600
BN = 1000


def _edge_core(F0, F1, F2, sp, w1f, b1, w2, b2, w3, b3, rep_m):
    Fs = (F0, F1, F2)
    Rs = [jnp.dot(Fk, rep_m, preferred_element_type=jnp.float32) for Fk in Fs]
    Ts = [jnp.concatenate([Fk] * 32, axis=1) for Fk in Fs]
    f2s = Rs[0] * Ts[0] + Rs[1] * Ts[1] + Rs[2] * Ts[2]
    nrm2 = jnp.sum(f2s * f2s, axis=1, keepdims=True)
    fnorm = jnp.sqrt(nrm2) + 1.0

    h = jnp.dot(f2s, w1f, preferred_element_type=jnp.float32) + sp + b1
    h = jnp.maximum(h, 0.0)
    h = jnp.dot(h, w2, preferred_element_type=jnp.float32) + b2
    h = jnp.maximum(h, 0.0)
    cvec = jnp.dot(h, w3, preferred_element_type=jnp.float32) + b3
    cvec = cvec * (1.0 / fnorm)

    cmain = cvec[:, :1024]
    outs = []
    for k in range(3):
        p = Rs[k] * cmain
        p = p[:, :512] + p[:, 512:]
        p = p[:, :256] + p[:, 256:]
        p = p[:, :128] + p[:, 128:]
        outs.append(p[:, 0:32] + p[:, 32:64] + p[:, 64:96] + p[:, 96:128])
    f_msg = jnp.concatenate(outs, axis=1)
    s_msg = cvec[:, 1024:1088]
    return f_msg, s_msg


def _onehot_mats():
    col = jnp.arange(1024, dtype=jnp.int32)
    row = jnp.arange(32, dtype=jnp.int32)[:, None]
    rep_m = (col[None, :] // 32 == row).astype(jnp.float32)
    return rep_m


def _blockdiag_mm(fb, w):
    return jnp.concatenate(
        [jnp.dot(fb[:, 32 * k:32 * (k + 1)], w,
                 preferred_element_type=jnp.float32) for k in range(3)],
        axis=1)


def _prep_body(f_ref, s_ref, we1_ref, w1_ref, u_ref, v_ref):
    fb = f_ref[...]
    sb = s_ref[...]
    we1 = we1_ref[...]
    a = _blockdiag_mm(fb, we1[:VD])
    b = _blockdiag_mm(fb, we1[VD:])
    ps = jnp.dot(sb, w1_ref[1024:1088], preferred_element_type=jnp.float32)
    qs = jnp.dot(sb, w1_ref[1088:1152], preferred_element_type=jnp.float32)
    z = jnp.zeros((fb.shape[0], DU - DU_USED), jnp.float32)
    u_ref[...] = jnp.concatenate([a, ps, z], axis=1)
    v_ref[...] = jnp.concatenate([b, qs, z], axis=1)


def _run_prep(f_flat, s, w_emb1, nw1):
    nb = N // BN
    full = lambda shp: pl.BlockSpec(shp, lambda i: (0, 0))
    return pl.pallas_call(
        _prep_body,
        grid=(nb,),
        in_specs=[
            pl.BlockSpec((BN, 96), lambda i: (i, 0)),
            pl.BlockSpec((BN, HD), lambda i: (i, 0)),
            full((2 * VD, VD)), full((DIN, HD)),
        ],
        out_specs=[
            pl.BlockSpec((BN, DU), lambda i: (i, 0)),
            pl.BlockSpec((BN, DU), lambda i: (i, 0)),
        ],
        out_shape=[
            jax.ShapeDtypeStruct((N, DU), jnp.float32),
            jax.ShapeDtypeStruct((N, DU), jnp.float32),
        ],
    )(f_flat, s, w_emb1, nw1)


G_STRIDE = 4992
G_NCH = 41


def _run_gather(u_tab, v_tab, src, dst):
    mesh = plsc.VectorSubcoreMesh(core_axis_name="c", subcore_axis_name="s")
    nidx = G_NCH * CH

    @functools.partial(
        pl.kernel,
        mesh=mesh,
        out_type=[
            jax.ShapeDtypeStruct((E, DU), jnp.float32),
            jax.ShapeDtypeStruct((E, DU), jnp.float32),
        ],
        scratch_types=[
            pltpu.VMEM((nidx,), jnp.int32),
            pltpu.VMEM((nidx,), jnp.int32),
            pltpu.VMEM((CH, DU), jnp.float32),
            pltpu.VMEM((CH, DU), jnp.float32),
            pltpu.SemaphoreType.DMA,
            pltpu.SemaphoreType.DMA,
            pltpu.SemaphoreType.DMA,
            pltpu.SemaphoreType.DMA,
        ],
    )
    def k(u_hbm, v_hbm, src_hbm, dst_hbm, gu_hbm, gv_hbm,
          idx_u, idx_d, urows, vrows, gsem_u, gsem_v, ssem_u, ssem_v):
        cid = lax.axis_index("c")
        sid = lax.axis_index("s")
        w = sid * NC + cid
        base_w = pl.multiple_of(w * G_STRIDE, 8)

        pltpu.sync_copy(src_hbm.at[pl.ds(base_w, nidx)], idx_u)
        pltpu.sync_copy(dst_hbm.at[pl.ds(base_w, nidx)], idx_d)

        def start(ci):
            lo = pl.multiple_of(ci * CH, 8)
            pltpu.async_copy(u_hbm.at[idx_u.at[pl.ds(lo, CH)]],
                             urows, gsem_u)
            pltpu.async_copy(v_hbm.at[idx_d.at[pl.ds(lo, CH)]],
                             vrows, gsem_v)

        start(0)

        def body(ci, carry):
            base = pl.multiple_of(base_w + ci * CH, 8)
            lo = pl.multiple_of(ci * CH, 8)
            pltpu.make_async_copy(u_hbm.at[idx_u.at[pl.ds(lo, CH)]],
                                  urows, gsem_u).wait()
            su = pltpu.async_copy(urows, gu_hbm.at[pl.ds(base, CH)], ssem_u)
            pltpu.make_async_copy(v_hbm.at[idx_d.at[pl.ds(lo, CH)]],
                                  vrows, gsem_v).wait()
            sv = pltpu.async_copy(vrows, gv_hbm.at[pl.ds(base, CH)], ssem_v)
            su.wait()
            sv.wait()

            @pl.when(ci < G_NCH - 1)
            def _():
                start(ci + 1)

            return carry

        lax.fori_loop(0, G_NCH, body, 0)

    return k(u_tab, v_tab, src, dst)


def _edge_body(gu_ref, gv_ref, w1_ref, b1_ref, w2_ref, b2_ref, w3_ref,
               b3_ref, rep_ref, o_ref):
    u = gu_ref[...]
    v = gv_ref[...]
    fvec = u[:, :96] + v[:, :96]
    sp = u[:, 96:DU_USED] + v[:, 96:DU_USED]
    f_msg, s_msg = _edge_core(
        fvec[:, 0:32], fvec[:, 32:64], fvec[:, 64:96], sp,
        w1_ref[:1024], b1_ref[...], w2_ref[...], b2_ref[...],
        w3_ref[...], b3_ref[...], rep_ref[...])
    B = f_msg.shape[0]
    ones = jnp.ones((B, 16), jnp.float32)
    zpad = jnp.zeros((B, DO - DU_USED - 16), jnp.float32)
    o_ref[...] = jnp.concatenate([f_msg, s_msg, ones, zpad], axis=1)


def _run_edge(gu, gv, w1, b1, w2, b2, w3, b3, rep_m):
    nb = E // BE
    full = lambda shp: pl.BlockSpec(shp, lambda i: tuple(0 for _ in shp))
    return pl.pallas_call(
        _edge_body,
        grid=(nb,),
        in_specs=[
            pl.BlockSpec((BE, DU), lambda i: (i, 0)),
            pl.BlockSpec((BE, DU), lambda i: (i, 0)),
            full((DIN, HD)), full((1, HD)), full((HD, HD)), full((1, HD)),
            full((HD, 1088)), full((1, 1088)),
            full((VD, 1024)),
        ],
        out_specs=pl.BlockSpec((BE, DO), lambda i: (i, 0)),
        out_shape=jax.ShapeDtypeStruct((E, DO), jnp.float32),
    )(gu, gv, w1, b1, w2, b2, w3, b3, rep_m)


HW = DO // NC
N_ACC = 10240


def _run_scatter(o_rows, src, zeros_init):
    mesh = plsc.VectorSubcoreMesh(core_axis_name="c", subcore_axis_name="s")
    nfull = EPT // CH
    tail = EPT - nfull * CH
    rows_per_tile = N_ACC // NS

    @functools.partial(
        pl.kernel,
        mesh=mesh,
        out_type=jax.ShapeDtypeStruct((N_ACC, DO), jnp.float32),
        scratch_types=[
            pltpu.VMEM((CH,), jnp.int32),
            pltpu.VMEM((CH,), jnp.int32),
            pltpu.VMEM((CH, HW), jnp.float32),
            pltpu.VMEM((CH, HW), jnp.float32),
            pltpu.VMEM((tail,), jnp.int32),
            pltpu.VMEM((tail, HW), jnp.float32),
            pltpu.VMEM_SHARED((N_ACC, HW), jnp.float32),
            pltpu.SemaphoreType.DMA,
            pltpu.SemaphoreType.DMA,
            pltpu.SemaphoreType.DMA,
            pltpu.SemaphoreType.DMA,
        ],
    )
    def k(o_hbm, src_hbm, z_hbm, out_hbm, idx0, idx1, rows0, rows1,
          idx_t, rows_t, accum, isem0, isem1, rsem0, rsem1):
        cid = lax.axis_index("c")
        sid = lax.axis_index("s")
        base_w = sid * EPT
        coff = pl.multiple_of(cid * HW, HW)
        r0 = pl.multiple_of(sid * rows_per_tile, 8)
        bufs = ((idx0, rows0, isem0, rsem0), (idx1, rows1, isem1, rsem1))

        pltpu.sync_copy(z_hbm, accum.at[pl.ds(r0, rows_per_tile)])
        plsc.subcore_barrier()

        def start(ci, ib, rb, isem, rsem):
            base = pl.multiple_of(base_w + ci * CH, 8)
            pltpu.async_copy(src_hbm.at[pl.ds(base, CH)], ib, isem)
            pltpu.async_copy(o_hbm.at[pl.ds(base, CH), pl.ds(coff, HW)],
                             rb, rsem)

        for b in range(2):
            start(b, *bufs[b])

        def body(m, carry):
            for b in range(2):
                ci = 2 * m + b
                ib, rb, isem, rsem = bufs[b]
                base = pl.multiple_of(base_w + ci * CH, 8)
                pltpu.make_async_copy(src_hbm.at[pl.ds(base, CH)],
                                      ib, isem).wait()
                pltpu.make_async_copy(
                    o_hbm.at[pl.ds(base, CH), pl.ds(coff, HW)],
                    rb, rsem).wait()
                pltpu.sync_copy(rb, accum.at[ib], add=True)

                @pl.when(ci + 2 < nfull)
                def _():
                    start(ci + 2, ib, rb, isem, rsem)

            return carry

        lax.fori_loop(0, nfull // 2, body, 0)

        bt = pl.multiple_of(base_w + nfull * CH, 8)
        pltpu.sync_copy(src_hbm.at[pl.ds(bt, tail)], idx_t)
        pltpu.sync_copy(o_hbm.at[pl.ds(bt, tail), pl.ds(coff, HW)], rows_t)
        pltpu.sync_copy(rows_t, accum.at[idx_t], add=True)

        plsc.subcore_barrier()
        pltpu.sync_copy(accum.at[pl.ds(r0, rows_per_tile)],
                        out_hbm.at[pl.ds(r0, rows_per_tile),
                                   pl.ds(coff, HW)])

    return k(o_rows, src, zeros_init)


def _node_body(f_ref, s_ref, p_ref, we2_ref, sw1_ref, b1_ref, w2_ref,
               b2_ref, w3_ref, b3_ref, rep_ref, fo_ref, so_ref):
    fb = f_ref[...]
    sb = s_ref[...]
    ps = p_ref[...]
    cnt = jnp.maximum(ps[:, 160:161], 1.0)
    inv = 1.0 / cnt
    f_c = ps[:, :96] * inv
    s_c = ps[:, 96:160] * inv

    we2 = we2_ref[...]
    tf = _blockdiag_mm(fb, we2[:VD])
    tf = tf + _blockdiag_mm(f_c, we2[VD:])
    sp = (jnp.dot(sb, sw1_ref[1024:1088], preferred_element_type=jnp.float32)
          + jnp.dot(s_c, sw1_ref[1088:1152],
                    preferred_element_type=jnp.float32))
    f_msg, s_msg = _edge_core(
        tf[:, 0:32], tf[:, 32:64], tf[:, 64:96], sp,
        sw1_ref[:1024], b1_ref[...], w2_ref[...], b2_ref[...],
        w3_ref[...], b3_ref[...], rep_ref[...])
    fo_ref[...] = f_msg + fb
    so_ref[...] = s_msg + sb


def _run_node(f_flat, s, partials, w_emb2, sw1, sb1, sw2, sb2, sw3, sb3,
              rep_m):
    nb = N // BN
    full = lambda shp: pl.BlockSpec(shp, lambda i: tuple(0 for _ in shp))
    return pl.pallas_call(
        _node_body,
        grid=(nb,),
        in_specs=[
            pl.BlockSpec((BN, 96), lambda i: (i, 0)),
            pl.BlockSpec((BN, HD), lambda i: (i, 0)),
            pl.BlockSpec((BN, DO), lambda i: (i, 0)),
            full((2 * VD, VD)), full((DIN, HD)), full((1, HD)),
            full((HD, HD)), full((1, HD)), full((HD, 1088)), full((1, 1088)),
            full((VD, 1024)),
        ],
        out_specs=[
            pl.BlockSpec((BN, 96), lambda i: (i, 0)),
            pl.BlockSpec((BN, HD), lambda i: (i, 0)),
        ],
        out_shape=[
            jax.ShapeDtypeStruct((N, 96), jnp.float32),
            jax.ShapeDtypeStruct((N, HD), jnp.float32),
        ],
    )(f_flat, s, partials, w_emb2, sw1, sb1, sw2, sb2, sw3, sb3, rep_m)


def kernel(f, s, edge_index, W_emb1, W_emb2, nW1, nb1, nW2, nb2, nW3, nb3,
           sW1, sb1, sW2, sb2, sW3, sb3):
    f_flat = f.reshape(N, 96)
    src = edge_index[0].astype(jnp.int32)
    dst = edge_index[1].astype(jnp.int32)

    u_tab, v_tab = _run_prep(f_flat, s, W_emb1, nW1)
    gu, gv = _run_gather(u_tab, v_tab, src, dst)
    rep_m = _onehot_mats()
    o_rows = _run_edge(gu, gv, nW1, nb1.reshape(1, HD), nW2,
                       nb2.reshape(1, HD), nW3, nb3.reshape(1, 1088),
                       rep_m)
    zeros_init = jnp.zeros((N_ACC // NS, HW), jnp.float32)
    partials = _run_scatter(o_rows, src, zeros_init)[:N]
    f_out, s_out = _run_node(f_flat, s, partials, W_emb2, sW1,
                             sb1.reshape(1, HD), sW2, sb2.reshape(1, HD),
                             sW3, sb3.reshape(1, 1088), rep_m)
    return (f_out.reshape(N, 3, VD), s_out)


# speedup vs baseline: 19.2792x; 1.1300x over previous
import functools

import jax
import jax.numpy as jnp
from jax import lax
from jax.experimental import pallas as pl
from jax.experimental.pallas import tpu as pltpu
from jax.experimental.pallas import tpu_sc as plsc

VD = 32
HD = 64
N = 10000
E = 160000
DIN = VD * VD + 2 * HD
DU = 256
DO = 256

DU_USED = 160
NC, NS = 2, 16
NW = NC * NS
EPW = E // NW
EPT = E // NS
CH = 128

BE = 1600
BN = 1000


def _edge_core(F0, F1, F2, sp, w1f, b1, w2, b2, w3, b3, rep_m):
    Fs = (F0, F1, F2)
    Rs = [jnp.dot(Fk, rep_m, preferred_element_type=jnp.float32) for Fk in Fs]
    Ts = [jnp.concatenate([Fk] * 32, axis=1) for Fk in Fs]
    f2s = Rs[0] * Ts[0] + Rs[1] * Ts[1] + Rs[2] * Ts[2]
    nrm2 = jnp.sum(f2s * f2s, axis=1, keepdims=True)
    fnorm = jnp.sqrt(nrm2) + 1.0

    h = jnp.dot(f2s, w1f, preferred_element_type=jnp.float32) + sp + b1
    h = jnp.maximum(h, 0.0)
    h = jnp.dot(h, w2, preferred_element_type=jnp.float32) + b2
    h = jnp.maximum(h, 0.0)
    cvec = jnp.dot(h, w3, preferred_element_type=jnp.float32) + b3
    cvec = cvec * (1.0 / fnorm)

    cmain = cvec[:, :1024]
    outs = []
    for k in range(3):
        p = Rs[k] * cmain
        p = p[:, :512] + p[:, 512:]
        p = p[:, :256] + p[:, 256:]
        p = p[:, :128] + p[:, 128:]
        outs.append(p[:, 0:32] + p[:, 32:64] + p[:, 64:96] + p[:, 96:128])
    f_msg = jnp.concatenate(outs, axis=1)
    s_msg = cvec[:, 1024:1088]
    return f_msg, s_msg


def _onehot_mats():
    col = jnp.arange(1024, dtype=jnp.int32)
    row = jnp.arange(32, dtype=jnp.int32)[:, None]
    rep_m = (col[None, :] // 32 == row).astype(jnp.float32)
    return rep_m


def _blockdiag_mm(fb, w):
    return jnp.concatenate(
        [jnp.dot(fb[:, 32 * k:32 * (k + 1)], w,
                 preferred_element_type=jnp.float32) for k in range(3)],
        axis=1)


def _prep_body(f_ref, s_ref, we1_ref, w1_ref, u_ref, v_ref):
    fb = f_ref[...]
    sb = s_ref[...]
    we1 = we1_ref[...]
    a = _blockdiag_mm(fb, we1[:VD])
    b = _blockdiag_mm(fb, we1[VD:])
    ps = jnp.dot(sb, w1_ref[1024:1088], preferred_element_type=jnp.float32)
    qs = jnp.dot(sb, w1_ref[1088:1152], preferred_element_type=jnp.float32)
    z = jnp.zeros((fb.shape[0], DU - DU_USED), jnp.float32)
    u_ref[...] = jnp.concatenate([a, ps, z], axis=1)
    v_ref[...] = jnp.concatenate([b, qs, z], axis=1)


def _run_prep(f_flat, s, w_emb1, nw1):
    nb = N // BN
    full = lambda shp: pl.BlockSpec(shp, lambda i: (0, 0))
    return pl.pallas_call(
        _prep_body,
        grid=(nb,),
        in_specs=[
            pl.BlockSpec((BN, 96), lambda i: (i, 0)),
            pl.BlockSpec((BN, HD), lambda i: (i, 0)),
            full((2 * VD, VD)), full((DIN, HD)),
        ],
        out_specs=[
            pl.BlockSpec((BN, DU), lambda i: (i, 0)),
            pl.BlockSpec((BN, DU), lambda i: (i, 0)),
        ],
        out_shape=[
            jax.ShapeDtypeStruct((N, DU), jnp.float32),
            jax.ShapeDtypeStruct((N, DU), jnp.float32),
        ],
    )(f_flat, s, w_emb1, nw1)


EH = E // 2
G_STRIDE = 2560
G_NCH = 20


def _run_gather(u_tab, v_tab, src, dst, e0):
    mesh = plsc.VectorSubcoreMesh(core_axis_name="c", subcore_axis_name="s")
    nidx = G_NCH * CH

    @functools.partial(
        pl.kernel,
        mesh=mesh,
        out_type=[
            jax.ShapeDtypeStruct((EH, DU), jnp.float32),
            jax.ShapeDtypeStruct((EH, DU), jnp.float32),
        ],
        scratch_types=[
            pltpu.VMEM((nidx,), jnp.int32),
            pltpu.VMEM((nidx,), jnp.int32),
            pltpu.VMEM((CH, DU), jnp.float32),
            pltpu.VMEM((CH, DU), jnp.float32),
            pltpu.SemaphoreType.DMA,
            pltpu.SemaphoreType.DMA,
            pltpu.SemaphoreType.DMA,
            pltpu.SemaphoreType.DMA,
        ],
    )
    def k(u_hbm, v_hbm, src_hbm, dst_hbm, gu_hbm, gv_hbm,
          idx_u, idx_d, urows, vrows, gsem_u, gsem_v, ssem_u, ssem_v):
        cid = lax.axis_index("c")
        sid = lax.axis_index("s")
        w = sid * NC + cid
        base_w = pl.multiple_of(jnp.minimum(w * G_STRIDE, EH - nidx), 8)

        pltpu.sync_copy(src_hbm.at[pl.ds(base_w + e0, nidx)], idx_u)
        pltpu.sync_copy(dst_hbm.at[pl.ds(base_w + e0, nidx)], idx_d)

        def start(ci):
            lo = pl.multiple_of(ci * CH, 8)
            pltpu.async_copy(u_hbm.at[idx_u.at[pl.ds(lo, CH)]],
                             urows, gsem_u)
            pltpu.async_copy(v_hbm.at[idx_d.at[pl.ds(lo, CH)]],
                             vrows, gsem_v)

        start(0)

        def body(ci, carry):
            base = pl.multiple_of(base_w + ci * CH, 8)
            lo = pl.multiple_of(ci * CH, 8)
            pltpu.make_async_copy(u_hbm.at[idx_u.at[pl.ds(lo, CH)]],
                                  urows, gsem_u).wait()
            su = pltpu.async_copy(urows, gu_hbm.at[pl.ds(base, CH)], ssem_u)
            pltpu.make_async_copy(v_hbm.at[idx_d.at[pl.ds(lo, CH)]],
                                  vrows, gsem_v).wait()
            sv = pltpu.async_copy(vrows, gv_hbm.at[pl.ds(base, CH)], ssem_v)
            su.wait()
            sv.wait()

            @pl.when(ci < G_NCH - 1)
            def _():
                start(ci + 1)

            return carry

        lax.fori_loop(0, G_NCH, body, 0)

    return k(u_tab, v_tab, src, dst)


def _edge_body(gu_ref, gv_ref, w1_ref, b1_ref, w2_ref, b2_ref, w3_ref,
               b3_ref, rep_ref, o_ref):
    u = gu_ref[...]
    v = gv_ref[...]
    fvec = u[:, :96] + v[:, :96]
    sp = u[:, 96:DU_USED] + v[:, 96:DU_USED]
    f_msg, s_msg = _edge_core(
        fvec[:, 0:32], fvec[:, 32:64], fvec[:, 64:96], sp,
        w1_ref[:1024], b1_ref[...], w2_ref[...], b2_ref[...],
        w3_ref[...], b3_ref[...], rep_ref[...])
    B = f_msg.shape[0]
    ones = jnp.ones((B, 16), jnp.float32)
    zpad = jnp.zeros((B, DO - DU_USED - 16), jnp.float32)
    o_ref[...] = jnp.concatenate([f_msg, s_msg, ones, zpad], axis=1)


def _run_edge(gu, gv, w1, b1, w2, b2, w3, b3, rep_m):
    nb = EH // BE
    full = lambda shp: pl.BlockSpec(shp, lambda i: tuple(0 for _ in shp))
    return pl.pallas_call(
        _edge_body,
        grid=(nb,),
        in_specs=[
            pl.BlockSpec((BE, DU), lambda i: (i, 0)),
            pl.BlockSpec((BE, DU), lambda i: (i, 0)),
            full((DIN, HD)), full((1, HD)), full((HD, HD)), full((1, HD)),
            full((HD, 1088)), full((1, 1088)),
            full((VD, 1024)),
        ],
        out_specs=pl.BlockSpec((BE, DO), lambda i: (i, 0)),
        out_shape=jax.ShapeDtypeStruct((EH, DO), jnp.float32),
    )(gu, gv, w1, b1, w2, b2, w3, b3, rep_m)


HW = DO // NC
N_ACC = 10240


def _run_scatter(o_rows, src, zeros_init, e0):
    mesh = plsc.VectorSubcoreMesh(core_axis_name="c", subcore_axis_name="s")
    ept = EH // NS
    nfull = ept // CH
    tail = ept - nfull * CH
    rows_per_tile = N_ACC // NS

    @functools.partial(
        pl.kernel,
        mesh=mesh,
        out_type=jax.ShapeDtypeStruct((N_ACC, DO), jnp.float32),
        scratch_types=[
            pltpu.VMEM((CH,), jnp.int32),
            pltpu.VMEM((CH,), jnp.int32),
            pltpu.VMEM((CH, HW), jnp.float32),
            pltpu.VMEM((CH, HW), jnp.float32),
            pltpu.VMEM((tail,), jnp.int32),
            pltpu.VMEM((tail, HW), jnp.float32),
            pltpu.VMEM_SHARED((N_ACC, HW), jnp.float32),
            pltpu.SemaphoreType.DMA,
            pltpu.SemaphoreType.DMA,
            pltpu.SemaphoreType.DMA,
            pltpu.SemaphoreType.DMA,
        ],
    )
    def k(o_hbm, src_hbm, z_hbm, out_hbm, idx0, idx1, rows0, rows1,
          idx_t, rows_t, accum, isem0, isem1, rsem0, rsem1):
        cid = lax.axis_index("c")
        sid = lax.axis_index("s")
        base_w = sid * ept
        coff = pl.multiple_of(cid * HW, HW)
        r0 = pl.multiple_of(sid * rows_per_tile, 8)
        bufs = ((idx0, rows0, isem0, rsem0), (idx1, rows1, isem1, rsem1))

        pltpu.sync_copy(z_hbm, accum.at[pl.ds(r0, rows_per_tile)])
        plsc.subcore_barrier()

        def start(ci, ib, rb, isem, rsem):
            base = pl.multiple_of(base_w + ci * CH, 8)
            pltpu.async_copy(src_hbm.at[pl.ds(base + e0, CH)], ib, isem)
            pltpu.async_copy(o_hbm.at[pl.ds(base, CH), pl.ds(coff, HW)],
                             rb, rsem)

        for b in range(2):
            start(b, *bufs[b])

        def body(m, carry):
            for b in range(2):
                ci = 2 * m + b
                ib, rb, isem, rsem = bufs[b]
                base = pl.multiple_of(base_w + ci * CH, 8)
                pltpu.make_async_copy(src_hbm.at[pl.ds(base + e0, CH)],
                                      ib, isem).wait()
                pltpu.make_async_copy(
                    o_hbm.at[pl.ds(base, CH), pl.ds(coff, HW)],
                    rb, rsem).wait()
                pltpu.sync_copy(rb, accum.at[ib], add=True)

                @pl.when(ci + 2 < nfull)
                def _():
                    start(ci + 2, ib, rb, isem, rsem)

            return carry

        lax.fori_loop(0, nfull // 2, body, 0)

        if nfull % 2 == 1:
            ci = nfull - 1
            ib, rb, isem, rsem = bufs[ci % 2]
            base = pl.multiple_of(base_w + ci * CH, 8)
            pltpu.make_async_copy(src_hbm.at[pl.ds(base + e0, CH)],
                                  ib, isem).wait()
            pltpu.make_async_copy(
                o_hbm.at[pl.ds(base, CH), pl.ds(coff, HW)], rb, rsem).wait()
            pltpu.sync_copy(rb, accum.at[ib], add=True)

        bt = pl.multiple_of(base_w + nfull * CH, 8)
        pltpu.sync_copy(src_hbm.at[pl.ds(bt + e0, tail)], idx_t)
        pltpu.sync_copy(o_hbm.at[pl.ds(bt, tail), pl.ds(coff, HW)], rows_t)
        pltpu.sync_copy(rows_t, accum.at[idx_t], add=True)

        plsc.subcore_barrier()
        pltpu.sync_copy(accum.at[pl.ds(r0, rows_per_tile)],
                        out_hbm.at[pl.ds(r0, rows_per_tile),
                                   pl.ds(coff, HW)])

    return k(o_rows, src, zeros_init)


def _node_body(f_ref, s_ref, p_ref, q_ref, we2_ref, sw1_ref, b1_ref, w2_ref,
               b2_ref, w3_ref, b3_ref, rep_ref, fo_ref, so_ref):
    fb = f_ref[...]
    sb = s_ref[...]
    ps = p_ref[...] + q_ref[...]
    cnt = jnp.maximum(ps[:, 160:161], 1.0)
    inv = 1.0 / cnt
    f_c = ps[:, :96] * inv
    s_c = ps[:, 96:160] * inv

    we2 = we2_ref[...]
    tf = _blockdiag_mm(fb, we2[:VD])
    tf = tf + _blockdiag_mm(f_c, we2[VD:])
    sp = (jnp.dot(sb, sw1_ref[1024:1088], preferred_element_type=jnp.float32)
          + jnp.dot(s_c, sw1_ref[1088:1152],
                    preferred_element_type=jnp.float32))
    f_msg, s_msg = _edge_core(
        tf[:, 0:32], tf[:, 32:64], tf[:, 64:96], sp,
        sw1_ref[:1024], b1_ref[...], w2_ref[...], b2_ref[...],
        w3_ref[...], b3_ref[...], rep_ref[...])
    fo_ref[...] = f_msg + fb
    so_ref[...] = s_msg + sb


def _run_node(f_flat, s, part0, part1, w_emb2, sw1, sb1, sw2, sb2, sw3, sb3,
              rep_m):
    nb = N // BN
    full = lambda shp: pl.BlockSpec(shp, lambda i: tuple(0 for _ in shp))
    return pl.pallas_call(
        _node_body,
        grid=(nb,),
        in_specs=[
            pl.BlockSpec((BN, 96), lambda i: (i, 0)),
            pl.BlockSpec((BN, HD), lambda i: (i, 0)),
            pl.BlockSpec((BN, DO), lambda i: (i, 0)),
            pl.BlockSpec((BN, DO), lambda i: (i, 0)),
            full((2 * VD, VD)), full((DIN, HD)), full((1, HD)),
            full((HD, HD)), full((1, HD)), full((HD, 1088)), full((1, 1088)),
            full((VD, 1024)),
        ],
        out_specs=[
            pl.BlockSpec((BN, 96), lambda i: (i, 0)),
            pl.BlockSpec((BN, HD), lambda i: (i, 0)),
        ],
        out_shape=[
            jax.ShapeDtypeStruct((N, 96), jnp.float32),
            jax.ShapeDtypeStruct((N, HD), jnp.float32),
        ],
    )(f_flat, s, part0, part1, w_emb2, sw1, sb1, sw2, sb2, sw3, sb3, rep_m)


def kernel(f, s, edge_index, W_emb1, W_emb2, nW1, nb1, nW2, nb2, nW3, nb3,
           sW1, sb1, sW2, sb2, sW3, sb3):
    f_flat = f.reshape(N, 96)
    src = edge_index[0].astype(jnp.int32)
    dst = edge_index[1].astype(jnp.int32)

    u_tab, v_tab = _run_prep(f_flat, s, W_emb1, nW1)
    rep_m = _onehot_mats()
    zeros_init = jnp.zeros((N_ACC // NS, HW), jnp.float32)
    ew = (nW1, nb1.reshape(1, HD), nW2, nb2.reshape(1, HD), nW3,
          nb3.reshape(1, 1088), rep_m)
    gu0, gv0 = _run_gather(u_tab, v_tab, src, dst, 0)
    gu1, gv1 = _run_gather(u_tab, v_tab, src, dst, EH)
    o0 = _run_edge(gu0, gv0, *ew)
    p0 = _run_scatter(o0, src, zeros_init, 0)[:N]
    o1 = _run_edge(gu1, gv1, *ew)
    p1 = _run_scatter(o1, src, zeros_init, EH)[:N]
    f_out, s_out = _run_node(f_flat, s, p0, p1, W_emb2, sW1,
                             sb1.reshape(1, HD), sW2, sb2.reshape(1, HD),
                             sW3, sb3.reshape(1, 1088), rep_m)
    return (f_out.reshape(N, 3, VD), s_out)
